# baseline (speedup 1.0000x reference)
import jax
import jax.numpy as jnp
from jax import lax
from jax.experimental import pallas as pl
from jax.experimental.pallas import tpu as pltpu
from jax.experimental.pallas import tpu_sc as plsc

N = 10000
E = 320000
D = 128
DE = 16
L = 16
NS = 16
PAD_N = 10240
EC = E // NS
B = 128
HALF = PAD_N // 2
ACC_R = HALF + 64
HSLICE = HALF // NS
CHK = 2048
NCHK = (EC + CHK - 1) // CHK
TAIL = EC - (NCHK - 1) * CHK
VPC = CHK // L
ET_B = 64
BLK = 1000
GRID = N // BLK


def _pre_body(xp, xg, xs, wpg, wps, wgp, wsp, ypg, yps, ygp, ysp):
    f32 = jnp.float32
    ypg[...] = jnp.dot(xp[...], wpg[...], preferred_element_type=f32)
    yps[...] = jnp.dot(xp[...], wps[...], preferred_element_type=f32)
    ygp[...] = jnp.dot(xg[...], wgp[...], preferred_element_type=f32)
    ysp[...] = jnp.dot(xs[...], wsp[...], preferred_element_type=f32)


def _pre(xp, xg, xs, wpg, wps, wgp, wsp):
    row_spec = pl.BlockSpec((BLK, D), lambda i: (i, 0))
    w_spec = pl.BlockSpec((D, D), lambda i: (0, 0))
    return pl.pallas_call(
        _pre_body,
        grid=(GRID,),
        in_specs=[row_spec, row_spec, row_spec, w_spec, w_spec, w_spec,
                  w_spec],
        out_specs=[row_spec] * 4,
        out_shape=[jax.ShapeDtypeStruct((N, D), jnp.float32)] * 4,
    )(xp, xg, xs, wpg, wps, wgp, wsp)


def _sc_body(y_pg, y_ps, y_gp, y_sp,
             er_pg, ec_pg, er_ps, ec_ps, er_gp, ec_gp, er_sp, ec_sp,
             ea_pg, ea_gp,
             sum_pg, sum_ps, sum_gp, sum_sp, cnt_pg, cnt_ps, cnt_gp, cnt_sp,
             ets_pg, ets_gp,
             rowchk, colchk, rowbuf, rowidx, colbuf, pendrow, pendcol,
             cntloc, winloc, redcnt, redwin, cntred, win8buf, wmodbuf,
             adjbuf, etraw, etflat,
             acc, cntsh, winsh,
             crsem0, crsem1, ccsem0, ccsem1, gsem0, gsem1, ssem0, ssem1,
             esem):
    c = lax.axis_index("c")
    s = lax.axis_index("s")
    i32 = jnp.int32
    zf16 = jnp.zeros((L,), jnp.float32)
    of16 = jnp.ones((L,), jnp.float32)
    iota16 = lax.iota(i32, L)
    lo = c * HALF
    crsems = (crsem0, crsem1)
    ccsems = (ccsem0, ccsem1)
    gsems = (gsem0, gsem1)
    ssems = (ssem0, ssem1)

    def chunk_descs(er_hbm, ec_hbm, ch, p, sz=CHK):
        base2 = s * EC
        dr = pltpu.make_async_copy(
            er_hbm.at[pl.ds(base2 + ch * CHK, sz)],
            rowchk.at[pl.ds(p * CHK, sz)], crsems[p])
        dc = pltpu.make_async_copy(
            ec_hbm.at[pl.ds(base2 + ch * CHK, sz)],
            colchk.at[pl.ds(p * CHK, sz)], ccsems[p])
        return dr, dc

    def process(y_hbm, er_hbm, ec_hbm, sum_hbm, cnt_hbm, ea_hbm, ets_hbm):
        def zrow(r, _):
            for k in range(D // L):
                rowbuf[0, r, pl.ds(k * L, L)] = zf16
            return 0
        lax.fori_loop(0, B, zrow, 0)
        arows = ACC_R // NS
        a0 = s * arows
        pltpu.sync_copy(rowbuf.at[0], acc.at[pl.ds(a0, B)])
        pltpu.sync_copy(rowbuf.at[0], acc.at[pl.ds(a0 + B, B)])
        pltpu.sync_copy(rowbuf.at[0, pl.ds(0, arows - 2 * B)],
                        acc.at[pl.ds(a0 + 2 * B, arows - 2 * B)])

        m1_16 = jnp.full((L,), -1, i32)

        def initloc(i, _):
            cntloc[pl.ds(i * L, L)] = zf16
            winloc[pl.ds(i * L, L)] = m1_16
            return 0
        lax.fori_loop(0, HALF // L, initloc, 0)
        adjbuf[pl.ds(L, L)] = jnp.full((L,), -16, i32)

        plsc.subcore_barrier()

        def do_fire(slot, f):
            @pl.when(f >= 2)
            def _():
                pltpu.make_async_copy(
                    rowbuf.at[slot], acc.at[colbuf.at[slot]],
                    ssems[slot]).wait()

            for k in range(B // L):
                colbuf[slot, pl.ds(k * L, L)] = pendcol[pl.ds(k * L, L)]
                rowidx[slot, pl.ds(k * L, L)] = pendrow[pl.ds(k * L, L)]
            pendrow[pl.ds(0, L)] = pendrow[pl.ds(B, L)]
            pendcol[pl.ds(0, L)] = pendcol[pl.ds(B, L)]
            pltpu.async_copy(
                y_hbm.at[rowidx.at[slot]], rowbuf.at[slot], gsems[slot])

            @pl.when(f >= 1)
            def _():
                prev = 1 - slot
                pltpu.make_async_copy(
                    y_hbm.at[rowidx.at[prev]], rowbuf.at[prev],
                    gsems[prev]).wait()
                pltpu.async_copy(
                    rowbuf.at[prev], acc.at[colbuf.at[prev]], ssems[prev],
                    add=True)

        def scan_vreg(off, e0, cnt, f):
            cvec = colchk[pl.ds(off, L)]
            rvec = rowchk[pl.ds(off, L)]
            cl = cvec - lo
            m = jnp.logical_and(cl >= 0, cl < HALF)
            cls = jnp.where(m, cl, 0)
            plsc.addupdate_scatter(cntloc, [cls], of16, mask=m)
            if ea_hbm is not None:
                key = jnp.where(m, cls * L + iota16,
                                jnp.full((L,), -16, i32))
                skey, sval = plsc.sort_key_val(key, iota16)
                adjbuf[pl.ds(0, L)] = skey
                nxt = adjbuf[pl.ds(1, L)]
                scol = skey >> 4
                winmask = jnp.logical_and(scol != (nxt >> 4), scol >= 0)
                evec = e0 + sval
                plsc.store_scatter(winloc, [jnp.maximum(scol, 0)], evec,
                                   mask=winmask)
            pcv = plsc.all_reduce_population_count(m)
            pc = pcv[0]
            plsc.store_compressed(pendrow.at[pl.ds(cnt, L)], rvec, mask=m)
            plsc.store_compressed(pendcol.at[pl.ds(cnt, L)], cls, mask=m)
            cnt2 = cnt + pc
            fire = cnt2 >= B

            @pl.when(jnp.logical_and(fire, (f & 1) == 0))
            def _():
                do_fire(0, f)

            @pl.when(jnp.logical_and(fire, (f & 1) == 1))
            def _():
                do_fire(1, f)

            cnt3 = jnp.where(fire, cnt2 - B, cnt2)
            f2 = jnp.where(fire, f + 1, f)
            return cnt3, f2

        def scan_chunk(p, ch, cnt, f):
            def vloop(v, carry):
                cnt_, f_ = carry
                e0 = s * EC + ch * CHK + v * L
                return scan_vreg(p * CHK + v * L, e0, cnt_, f_)
            return lax.fori_loop(0, VPC, vloop, (cnt, f))

        dr, dc = chunk_descs(er_hbm, ec_hbm, 0, 0)
        dr.start()
        dc.start()

        def pairloop(q, carry):
            cnt, f = carry
            ch = 2 * q
            d0r, d0c = chunk_descs(er_hbm, ec_hbm, ch, 0)
            d0r.wait()
            d0c.wait()
            d1r, d1c = chunk_descs(er_hbm, ec_hbm, ch + 1, 1)
            d1r.start()
            d1c.start()
            cnt, f = scan_chunk(0, ch, cnt, f)
            d1r.wait()
            d1c.wait()
            d2r, d2c = chunk_descs(er_hbm, ec_hbm, ch + 2, 0)
            d2r.start()
            d2c.start()
            cnt, f = scan_chunk(1, ch + 1, cnt, f)
            return cnt, f

        cnt, f = lax.fori_loop(0, NCHK // 2 - 1, pairloop,
                               (jnp.int32(0), jnp.int32(0)))

        d0r, d0c = chunk_descs(er_hbm, ec_hbm, NCHK - 2, 0)
        d0r.wait()
        d0c.wait()
        d1r, d1c = chunk_descs(er_hbm, ec_hbm, NCHK - 1, 1, sz=TAIL)
        d1r.start()
        d1c.start()
        cnt, f = scan_chunk(0, NCHK - 2, cnt, f)
        d1r.wait()
        d1c.wait()
        m1pad = jnp.full((L,), -1, i32)
        for t in range((CHK - TAIL) // L):
            colchk[pl.ds(CHK + TAIL + t * L, L)] = m1pad
        cnt, f = scan_chunk(1, NCHK - 1, cnt, f)

        for k in range(B // L):
            pos = iota16 + k * L
            mm = pos < cnt
            pendcol[pl.ds(k * L, L)] = jnp.where(
                mm, pendcol[pl.ds(k * L, L)], jnp.full((L,), HALF, i32))
            pendrow[pl.ds(k * L, L)] = jnp.where(
                mm, pendrow[pl.ds(k * L, L)], jnp.zeros((L,), i32))
        lastp = f & 1

        @pl.when(lastp == 0)
        def _():
            do_fire(0, f)
            pltpu.make_async_copy(
                y_hbm.at[rowidx.at[0]], rowbuf.at[0], gsems[0]).wait()
            pltpu.sync_copy(rowbuf.at[0], acc.at[colbuf.at[0]], add=True)

            @pl.when(f >= 1)
            def _():
                pltpu.make_async_copy(
                    rowbuf.at[1], acc.at[colbuf.at[1]], ssems[1]).wait()

        @pl.when(lastp == 1)
        def _():
            do_fire(1, f)
            pltpu.make_async_copy(
                y_hbm.at[rowidx.at[1]], rowbuf.at[1], gsems[1]).wait()
            pltpu.sync_copy(rowbuf.at[1], acc.at[colbuf.at[1]], add=True)

            @pl.when(f >= 1)
            def _():
                pltpu.make_async_copy(
                    rowbuf.at[0], acc.at[colbuf.at[0]], ssems[0]).wait()

        plsc.subcore_barrier()
        off = s * HSLICE
        pltpu.sync_copy(acc.at[pl.ds(off, HSLICE)],
                        sum_hbm.at[pl.ds(lo + off, HSLICE)])

        pltpu.sync_copy(cntloc, cntsh.at[pl.ds(s * HALF, HALF)])
        if ea_hbm is not None:
            pltpu.sync_copy(winloc, winsh.at[pl.ds(s * HALF, HALF)])
        plsc.subcore_barrier()

        cdescs = []
        wdescs = []
        for t in range(NS):
            dce = pltpu.make_async_copy(
                cntsh.at[pl.ds(t * HALF + off, HSLICE)],
                redcnt.at[pl.ds(t * HSLICE, HSLICE)], crsem0)
            dce.start()
            cdescs.append(dce)
            if ea_hbm is not None:
                dwe = pltpu.make_async_copy(
                    winsh.at[pl.ds(t * HALF + off, HSLICE)],
                    redwin.at[pl.ds(t * HSLICE, HSLICE)], crsem1)
                dwe.start()
                wdescs.append(dwe)
        for dce in cdescs:
            dce.wait()
        for dwe in wdescs:
            dwe.wait()

        def redbody(m, _):
            cv = redcnt[pl.ds(m * L, L)]
            for t in range(1, NS):
                cv = cv + redcnt[pl.ds(t * HSLICE + m * L, L)]
            cntred[pl.ds(m * L, L)] = cv
            if ea_hbm is not None:
                wv = redwin[pl.ds(m * L, L)]
                for t in range(1, NS):
                    wv = jnp.maximum(wv, redwin[pl.ds(t * HSLICE + m * L, L)])
                wv = jnp.maximum(wv, 0)
                win8buf[pl.ds(m * L, L)] = wv >> 3
                wmodbuf[pl.ds(m * L, L)] = (wv & 7) * DE
            return 0

        lax.fori_loop(0, HSLICE // L, redbody, 0)
        pltpu.sync_copy(cntred, cnt_hbm.at[pl.ds(lo + off, HSLICE)])

        if ea_hbm is not None:
            for bb in range(HSLICE // ET_B):
                pltpu.async_copy(
                    ea_hbm.at[win8buf.at[pl.ds(bb * ET_B, ET_B)]], etraw,
                    esem).wait()

                def etloop(i, _):
                    offv = wmodbuf[pl.ds(bb * ET_B + i, L)]
                    etflat[pl.ds(i * DE, L)] = etraw[i, pl.ds(offv[0], L)]
                    return 0

                lax.fori_loop(0, ET_B, etloop, 0)
                pltpu.sync_copy(
                    etflat,
                    ets_hbm.at[pl.ds((lo + off + bb * ET_B) * DE,
                                     ET_B * DE)])

        plsc.subcore_barrier()

    process(y_pg, er_pg, ec_pg, sum_pg, cnt_pg, ea_pg, ets_pg)
    process(y_gp, er_gp, ec_gp, sum_gp, cnt_gp, ea_gp, ets_gp)
    process(y_ps, er_ps, ec_ps, sum_ps, cnt_ps, None, None)
    process(y_sp, er_sp, ec_sp, sum_sp, cnt_sp, None, None)


def _sc(y_pg, y_ps, y_gp, y_sp, ei_pg, ei_ps, ei_gp, ei_sp, ea_pg, ea_gp):
    f32 = jnp.float32
    i32 = jnp.int32
    out_type = (
        [jax.ShapeDtypeStruct((PAD_N, D), f32)] * 4
        + [jax.ShapeDtypeStruct((PAD_N,), f32)] * 4
        + [jax.ShapeDtypeStruct((PAD_N * DE,), f32)] * 2
    )
    scratch = [
        pltpu.VMEM((2 * CHK,), i32),
        pltpu.VMEM((2 * CHK,), i32),
        pltpu.VMEM((2, B, D), f32),
        pltpu.VMEM((2, B), i32),
        pltpu.VMEM((2, B), i32),
        pltpu.VMEM((B + L,), i32),
        pltpu.VMEM((B + L,), i32),
        pltpu.VMEM((HALF,), f32),
        pltpu.VMEM((HALF,), i32),
        pltpu.VMEM((NS * HSLICE,), f32),
        pltpu.VMEM((NS * HSLICE,), i32),
        pltpu.VMEM((HSLICE,), f32),
        pltpu.VMEM((HSLICE,), i32),
        pltpu.VMEM((HSLICE + L,), i32),
        pltpu.VMEM((2 * L,), i32),
        pltpu.VMEM((ET_B, D), f32),
        pltpu.VMEM((ET_B * DE,), f32),
        pltpu.VMEM_SHARED((ACC_R, D), f32),
        pltpu.VMEM_SHARED((NS * HALF,), f32),
        pltpu.VMEM_SHARED((NS * HALF,), i32),
        pltpu.SemaphoreType.DMA,
        pltpu.SemaphoreType.DMA,
        pltpu.SemaphoreType.DMA,
        pltpu.SemaphoreType.DMA,
        pltpu.SemaphoreType.DMA,
        pltpu.SemaphoreType.DMA,
        pltpu.SemaphoreType.DMA,
        pltpu.SemaphoreType.DMA,
        pltpu.SemaphoreType.DMA,
    ]
    mesh = plsc.VectorSubcoreMesh(core_axis_name="c", subcore_axis_name="s")
    fn = pl.kernel(_sc_body, out_type=out_type, mesh=mesh,
                   scratch_types=scratch,
                   compiler_params=pltpu.CompilerParams(
                       needs_layout_passes=False))
    return fn(y_pg, y_ps, y_gp, y_sp,
              ei_pg[0], ei_pg[1], ei_ps[0], ei_ps[1],
              ei_gp[0], ei_gp[1], ei_sp[0], ei_sp[1],
              ea_pg, ea_gp)


def _fin_body(spg, cpg, etspg, xg, wrpg, wepg, bepg, blpg,
              sgp, cgp, etsgp, xp, wrgp, wegp, begp, blgp,
              ssp, csp, wrsp, blsp,
              sps, cps, xs, wrps, blps,
              wlin, blin, pw,
              hp, gw, hs):
    f32 = jnp.float32

    def mean(sref, cref):
        return sref[...] / jnp.maximum(cref[...], 1.0)

    def etterm(etsref, weref, beref, cref):
        has = (cref[...] > 0.0).astype(f32)
        return (jnp.dot(etsref[...], weref[...], preferred_element_type=f32)
                + beref[...]) * has

    hgw = (mean(spg, cpg) + blpg[...]
           + jnp.dot(xg[...], wrpg[...], preferred_element_type=f32)
           + etterm(etspg, wepg, bepg, cpg))
    hgw = jnp.maximum(hgw, 0.0)
    g = jnp.dot(hgw, wlin[...], preferred_element_type=f32) + blin[...]
    gw[...] = jnp.where(g >= 0.0, g, pw[...] * g)

    hpf = (mean(sgp, cgp) + blgp[...]
           + jnp.dot(xp[...], wrgp[...], preferred_element_type=f32)
           + etterm(etsgp, wegp, begp, cgp)
           + mean(ssp, csp) + blsp[...]
           + jnp.dot(xp[...], wrsp[...], preferred_element_type=f32))
    hp[...] = jnp.maximum(hpf, 0.0)

    hsw = (mean(sps, cps) + blps[...]
           + jnp.dot(xs[...], wrps[...], preferred_element_type=f32))
    hs[...] = jnp.maximum(hsw, 0.0)


def _fin(spg, cpg, etspg, xg, wrpg, wepg, bepg, blpg,
         sgp, cgp, etsgp, xp, wrgp, wegp, begp, blgp,
         ssp, csp, wrsp, blsp,
         sps, cps, xs, wrps, blps,
         wlin, blin, pw):
    row = pl.BlockSpec((BLK, D), lambda i: (i, 0))
    col1 = pl.BlockSpec((BLK, 1), lambda i: (i, 0))
    ets = pl.BlockSpec((BLK, DE), lambda i: (i, 0))
    wdd = pl.BlockSpec((D, D), lambda i: (0, 0))
    wed = pl.BlockSpec((DE, D), lambda i: (0, 0))
    b1d = pl.BlockSpec((1, D), lambda i: (0, 0))
    wl = pl.BlockSpec((D, 1), lambda i: (0, 0))
    b11 = pl.BlockSpec((1, 1), lambda i: (0, 0))
    in_specs = [row, col1, ets, row, wdd, wed, b1d, b1d,
                row, col1, ets, row, wdd, wed, b1d, b1d,
                row, col1, wdd, b1d,
                row, col1, row, wdd, b1d,
                wl, b11, b11]
    out_specs = [row, col1, row]
    out_shape = [jax.ShapeDtypeStruct((N, D), jnp.float32),
                 jax.ShapeDtypeStruct((N, 1), jnp.float32),
                 jax.ShapeDtypeStruct((N, D), jnp.float32)]
    return pl.pallas_call(
        _fin_body, grid=(GRID,), in_specs=in_specs, out_specs=out_specs,
        out_shape=out_shape,
    )(spg, cpg, etspg, xg, wrpg, wepg, bepg, blpg,
      sgp, cgp, etsgp, xp, wrgp, wegp, begp, blgp,
      ssp, csp, wrsp, blsp,
      sps, cps, xs, wrps, blps,
      wlin, blin, pw)


def kernel(x_pfas, x_gw, x_sw,
           edge_index_pg, edge_index_gp, edge_index_ps, edge_index_sp,
           edge_attr_pg, edge_attr_gp,
           Wl_pg, bl_pg, Wr_pg, We_pg, be_pg,
           Wl_gp, bl_gp, Wr_gp, We_gp, be_gp,
           Wl_ps, bl_ps, Wr_ps,
           Wl_sp, bl_sp, Wr_sp,
           W_lin, b_lin, prelu_w):
    y_pg, y_ps, y_gp, y_sp = _pre(x_pfas, x_gw, x_sw,
                                  Wl_pg, Wl_ps, Wl_gp, Wl_sp)

    one = 1.0 + 0.0 * prelu_w[0]
    ea2_pg = edge_attr_pg.reshape(E * DE // D, D) * one
    ea2_gp = edge_attr_gp.reshape(E * DE // D, D) * one

    (s_pg, s_ps, s_gp, s_sp, c_pg, c_ps, c_gp, c_sp, ets_pg, ets_gp) = _sc(
        y_pg, y_ps, y_gp, y_sp,
        edge_index_pg, edge_index_ps, edge_index_gp, edge_index_sp,
        ea2_pg, ea2_gp)

    ets_pg = ets_pg.reshape(PAD_N, DE)
    ets_gp = ets_gp.reshape(PAD_N, DE)
    c_pg2 = c_pg.reshape(PAD_N, 1)
    c_ps2 = c_ps.reshape(PAD_N, 1)
    c_gp2 = c_gp.reshape(PAD_N, 1)
    c_sp2 = c_sp.reshape(PAD_N, 1)

    hp, gw, hs = _fin(
        s_pg, c_pg2, ets_pg, x_gw, Wr_pg, We_pg, be_pg.reshape(1, D),
        bl_pg.reshape(1, D),
        s_gp, c_gp2, ets_gp, x_pfas, Wr_gp, We_gp, be_gp.reshape(1, D),
        bl_gp.reshape(1, D),
        s_sp, c_sp2, Wr_sp, bl_sp.reshape(1, D),
        s_ps, c_ps2, x_sw, Wr_ps, bl_ps.reshape(1, D),
        W_lin, b_lin.reshape(1, 1), prelu_w.reshape(1, 1))

    return (hp, gw, hs)


# speedup vs baseline: 1.1648x; 1.1648x over previous
import jax
import jax.numpy as jnp
from jax import lax
from jax.experimental import pallas as pl
from jax.experimental.pallas import tpu as pltpu
from jax.experimental.pallas import tpu_sc as plsc

N = 10000
E = 320000
D = 128
DE = 16
L = 16
NS = 16
PAD_N = 10240
EC = E // NS
B = 128
HALF = PAD_N // 2
ACC_R = HALF + 64
HSLICE = HALF // NS
CHK = 2048
NCHK = (EC + CHK - 1) // CHK
TAIL = EC - (NCHK - 1) * CHK
VPC = CHK // L
ET_B = 64
BLK = 1000
GRID = N // BLK


def _pre_body(xp, xg, xs, wpg, wps, wgp, wsp, ypg, yps, ygp, ysp):
    f32 = jnp.float32
    ypg[...] = jnp.dot(xp[...], wpg[...], preferred_element_type=f32)
    yps[...] = jnp.dot(xp[...], wps[...], preferred_element_type=f32)
    ygp[...] = jnp.dot(xg[...], wgp[...], preferred_element_type=f32)
    ysp[...] = jnp.dot(xs[...], wsp[...], preferred_element_type=f32)


def _pre(xp, xg, xs, wpg, wps, wgp, wsp):
    row_spec = pl.BlockSpec((BLK, D), lambda i: (i, 0))
    w_spec = pl.BlockSpec((D, D), lambda i: (0, 0))
    return pl.pallas_call(
        _pre_body,
        grid=(GRID,),
        in_specs=[row_spec, row_spec, row_spec, w_spec, w_spec, w_spec,
                  w_spec],
        out_specs=[row_spec] * 4,
        out_shape=[jax.ShapeDtypeStruct((N, D), jnp.float32)] * 4,
    )(xp, xg, xs, wpg, wps, wgp, wsp)


def _sc_body(y_pg, y_ps, y_gp, y_sp,
             er_pg, ec_pg, er_ps, ec_ps, er_gp, ec_gp, er_sp, ec_sp,
             ea_pg, ea_gp,
             sum_pg, sum_ps, sum_gp, sum_sp, cnt_pg, cnt_ps, cnt_gp, cnt_sp,
             ets_pg, ets_gp,
             rowchk, colchk, rowbuf, rowidx, colbuf, pendrow, pendcol,
             cntloc, winloc, redcnt, redwin, cntred, win8buf, wmodbuf,
             adjbuf, etraw, etflat,
             acc, cntsh, winsh,
             crsem0, crsem1, ccsem0, ccsem1, gsem0, gsem1, ssem0, ssem1,
             esem):
    c = lax.axis_index("c")
    s = lax.axis_index("s")
    i32 = jnp.int32
    zf16 = jnp.zeros((L,), jnp.float32)
    of16 = jnp.ones((L,), jnp.float32)
    iota16 = lax.iota(i32, L)
    lo = c * HALF
    crsems = (crsem0, crsem1)
    ccsems = (ccsem0, ccsem1)
    gsems = (gsem0, gsem1)
    ssems = (ssem0, ssem1)

    def chunk_descs(er_hbm, ec_hbm, ch, p, sz=CHK):
        base2 = s * EC
        dr = pltpu.make_async_copy(
            er_hbm.at[pl.ds(base2 + ch * CHK, sz)],
            rowchk.at[pl.ds(p * CHK, sz)], crsems[p])
        dc = pltpu.make_async_copy(
            ec_hbm.at[pl.ds(base2 + ch * CHK, sz)],
            colchk.at[pl.ds(p * CHK, sz)], ccsems[p])
        return dr, dc

    def process(y_hbm, er_hbm, ec_hbm, sum_hbm, cnt_hbm, ea_hbm, ets_hbm):
        def zrow(r, _):
            for k in range(D // L):
                rowbuf[0, r, pl.ds(k * L, L)] = zf16
            return 0
        lax.fori_loop(0, B, zrow, 0)
        arows = ACC_R // NS
        a0 = s * arows
        pltpu.sync_copy(rowbuf.at[0], acc.at[pl.ds(a0, B)])
        pltpu.sync_copy(rowbuf.at[0], acc.at[pl.ds(a0 + B, B)])
        pltpu.sync_copy(rowbuf.at[0, pl.ds(0, arows - 2 * B)],
                        acc.at[pl.ds(a0 + 2 * B, arows - 2 * B)])

        m1_16 = jnp.full((L,), -1, i32)

        def initloc(i, _):
            cntloc[pl.ds(i * L, L)] = zf16
            winloc[pl.ds(i * L, L)] = m1_16
            return 0
        lax.fori_loop(0, HALF // L, initloc, 0)
        adjbuf[pl.ds(L, L)] = jnp.full((L,), -16, i32)

        plsc.subcore_barrier()

        def do_fire(slot, f):
            @pl.when(f >= 2)
            def _():
                pltpu.make_async_copy(
                    rowbuf.at[slot], acc.at[colbuf.at[slot]],
                    ssems[slot]).wait()

            for k in range(B // L):
                colbuf[slot, pl.ds(k * L, L)] = pendcol[pl.ds(k * L, L)]
                rowidx[slot, pl.ds(k * L, L)] = pendrow[pl.ds(k * L, L)]
            pendrow[pl.ds(0, L)] = pendrow[pl.ds(B, L)]
            pendcol[pl.ds(0, L)] = pendcol[pl.ds(B, L)]
            pltpu.async_copy(
                y_hbm.at[rowidx.at[slot]], rowbuf.at[slot], gsems[slot])

            @pl.when(f >= 1)
            def _():
                prev = 1 - slot
                pltpu.make_async_copy(
                    y_hbm.at[rowidx.at[prev]], rowbuf.at[prev],
                    gsems[prev]).wait()
                pltpu.async_copy(
                    rowbuf.at[prev], acc.at[colbuf.at[prev]], ssems[prev],
                    add=True)

        def scan_vreg(off, e0, cnt, f):
            cvec = colchk[pl.ds(off, L)]
            rvec = rowchk[pl.ds(off, L)]
            cl = cvec - lo
            m = jnp.logical_and(cl >= 0, cl < HALF)
            cls = jnp.where(m, cl, 0)
            plsc.addupdate_scatter(cntloc, [cls], of16, mask=m)
            if ea_hbm is not None:
                key = jnp.where(m, cls * L + iota16,
                                jnp.full((L,), -16, i32))
                skey, sval = plsc.sort_key_val(key, iota16)
                adjbuf[pl.ds(0, L)] = skey
                nxt = adjbuf[pl.ds(1, L)]
                scol = skey >> 4
                winmask = jnp.logical_and(scol != (nxt >> 4), scol >= 0)
                evec = e0 + sval
                plsc.store_scatter(winloc, [jnp.maximum(scol, 0)], evec,
                                   mask=winmask)
            pcv = plsc.all_reduce_population_count(m)
            pc = pcv[0]
            plsc.store_compressed(pendrow.at[pl.ds(cnt, L)], rvec, mask=m)
            plsc.store_compressed(pendcol.at[pl.ds(cnt, L)], cls, mask=m)
            cnt2 = cnt + pc
            fire = cnt2 >= B

            @pl.when(jnp.logical_and(fire, (f & 1) == 0))
            def _():
                do_fire(0, f)

            @pl.when(jnp.logical_and(fire, (f & 1) == 1))
            def _():
                do_fire(1, f)

            cnt3 = jnp.where(fire, cnt2 - B, cnt2)
            f2 = jnp.where(fire, f + 1, f)
            return cnt3, f2

        def scan_chunk(p, ch, cnt, f):
            def vloop(v, carry):
                cnt_, f_ = carry
                e0 = s * EC + ch * CHK + v * L
                return scan_vreg(p * CHK + v * L, e0, cnt_, f_)
            return lax.fori_loop(0, VPC, vloop, (cnt, f))

        dr, dc = chunk_descs(er_hbm, ec_hbm, 0, 0)
        dr.start()
        dc.start()

        def pairloop(q, carry):
            cnt, f = carry
            ch = 2 * q
            d0r, d0c = chunk_descs(er_hbm, ec_hbm, ch, 0)
            d0r.wait()
            d0c.wait()
            d1r, d1c = chunk_descs(er_hbm, ec_hbm, ch + 1, 1)
            d1r.start()
            d1c.start()
            cnt, f = scan_chunk(0, ch, cnt, f)
            d1r.wait()
            d1c.wait()
            d2r, d2c = chunk_descs(er_hbm, ec_hbm, ch + 2, 0)
            d2r.start()
            d2c.start()
            cnt, f = scan_chunk(1, ch + 1, cnt, f)
            return cnt, f

        cnt, f = lax.fori_loop(0, NCHK // 2 - 1, pairloop,
                               (jnp.int32(0), jnp.int32(0)))

        d0r, d0c = chunk_descs(er_hbm, ec_hbm, NCHK - 2, 0)
        d0r.wait()
        d0c.wait()
        d1r, d1c = chunk_descs(er_hbm, ec_hbm, NCHK - 1, 1, sz=TAIL)
        d1r.start()
        d1c.start()
        cnt, f = scan_chunk(0, NCHK - 2, cnt, f)
        d1r.wait()
        d1c.wait()
        m1pad = jnp.full((L,), -1, i32)
        for t in range((CHK - TAIL) // L):
            colchk[pl.ds(CHK + TAIL + t * L, L)] = m1pad
        cnt, f = scan_chunk(1, NCHK - 1, cnt, f)

        for k in range(B // L):
            pos = iota16 + k * L
            mm = pos < cnt
            pendcol[pl.ds(k * L, L)] = jnp.where(
                mm, pendcol[pl.ds(k * L, L)], jnp.full((L,), HALF, i32))
            pendrow[pl.ds(k * L, L)] = jnp.where(
                mm, pendrow[pl.ds(k * L, L)], jnp.zeros((L,), i32))
        lastp = f & 1

        @pl.when(lastp == 0)
        def _():
            do_fire(0, f)
            pltpu.make_async_copy(
                y_hbm.at[rowidx.at[0]], rowbuf.at[0], gsems[0]).wait()
            pltpu.sync_copy(rowbuf.at[0], acc.at[colbuf.at[0]], add=True)

            @pl.when(f >= 1)
            def _():
                pltpu.make_async_copy(
                    rowbuf.at[1], acc.at[colbuf.at[1]], ssems[1]).wait()

        @pl.when(lastp == 1)
        def _():
            do_fire(1, f)
            pltpu.make_async_copy(
                y_hbm.at[rowidx.at[1]], rowbuf.at[1], gsems[1]).wait()
            pltpu.sync_copy(rowbuf.at[1], acc.at[colbuf.at[1]], add=True)

            @pl.when(f >= 1)
            def _():
                pltpu.make_async_copy(
                    rowbuf.at[0], acc.at[colbuf.at[0]], ssems[0]).wait()

        plsc.subcore_barrier()
        off = s * HSLICE
        pltpu.sync_copy(acc.at[pl.ds(off, HSLICE)],
                        sum_hbm.at[pl.ds(lo + off, HSLICE)])

        pltpu.sync_copy(cntloc, cntsh.at[pl.ds(s * HALF, HALF)])
        if ea_hbm is not None:
            pltpu.sync_copy(winloc, winsh.at[pl.ds(s * HALF, HALF)])
        plsc.subcore_barrier()

        cdescs = []
        wdescs = []
        for t in range(NS):
            dce = pltpu.make_async_copy(
                cntsh.at[pl.ds(t * HALF + off, HSLICE)],
                redcnt.at[pl.ds(t * HSLICE, HSLICE)], crsem0)
            dce.start()
            cdescs.append(dce)
            if ea_hbm is not None:
                dwe = pltpu.make_async_copy(
                    winsh.at[pl.ds(t * HALF + off, HSLICE)],
                    redwin.at[pl.ds(t * HSLICE, HSLICE)], crsem1)
                dwe.start()
                wdescs.append(dwe)
        for dce in cdescs:
            dce.wait()
        for dwe in wdescs:
            dwe.wait()

        def redbody(m, _):
            cv = redcnt[pl.ds(m * L, L)]
            for t in range(1, NS):
                cv = cv + redcnt[pl.ds(t * HSLICE + m * L, L)]
            cntred[pl.ds(m * L, L)] = cv
            if ea_hbm is not None:
                wv = redwin[pl.ds(m * L, L)]
                for t in range(1, NS):
                    wv = jnp.maximum(wv, redwin[pl.ds(t * HSLICE + m * L, L)])
                wv = jnp.maximum(wv, 0)
                win8buf[pl.ds(m * L, L)] = wv >> 3
                wmodbuf[pl.ds(m * L, L)] = (wv & 7) * DE
            return 0

        lax.fori_loop(0, HSLICE // L, redbody, 0)
        pltpu.sync_copy(cntred, cnt_hbm.at[pl.ds(lo + off, HSLICE)])

        if ea_hbm is not None:
            for bb in range(HSLICE // ET_B):
                pltpu.async_copy(
                    ea_hbm.at[win8buf.at[pl.ds(bb * ET_B, ET_B)]], etraw,
                    esem).wait()

                def etloop(i, _):
                    offv = wmodbuf[pl.ds(bb * ET_B + i, L)]
                    etflat[pl.ds(i * DE, L)] = etraw[i, pl.ds(offv[0], L)]
                    return 0

                lax.fori_loop(0, ET_B, etloop, 0)
                pltpu.sync_copy(
                    etflat,
                    ets_hbm.at[pl.ds((lo + off + bb * ET_B) * DE,
                                     ET_B * DE)])

        plsc.subcore_barrier()

    process(y_pg, er_pg, ec_pg, sum_pg, cnt_pg, ea_pg, ets_pg)
    process(y_gp, er_gp, ec_gp, sum_gp, cnt_gp, ea_gp, ets_gp)
    process(y_ps, er_ps, ec_ps, sum_ps, cnt_ps, None, None)
    process(y_sp, er_sp, ec_sp, sum_sp, cnt_sp, None, None)


def _sc(y_pg, y_ps, y_gp, y_sp, ei_pg, ei_ps, ei_gp, ei_sp, ea_pg, ea_gp):
    f32 = jnp.float32
    i32 = jnp.int32
    out_type = (
        [jax.ShapeDtypeStruct((PAD_N, D), f32)] * 4
        + [jax.ShapeDtypeStruct((PAD_N,), f32)] * 4
        + [jax.ShapeDtypeStruct((PAD_N * DE,), f32)] * 2
    )
    scratch = [
        pltpu.VMEM((2 * CHK,), i32),
        pltpu.VMEM((2 * CHK,), i32),
        pltpu.VMEM((2, B, D), f32),
        pltpu.VMEM((2, B), i32),
        pltpu.VMEM((2, B), i32),
        pltpu.VMEM((B + L,), i32),
        pltpu.VMEM((B + L,), i32),
        pltpu.VMEM((HALF,), f32),
        pltpu.VMEM((HALF,), i32),
        pltpu.VMEM((NS * HSLICE,), f32),
        pltpu.VMEM((NS * HSLICE,), i32),
        pltpu.VMEM((HSLICE,), f32),
        pltpu.VMEM((HSLICE,), i32),
        pltpu.VMEM((HSLICE + L,), i32),
        pltpu.VMEM((2 * L,), i32),
        pltpu.VMEM((ET_B, D), f32),
        pltpu.VMEM((ET_B * DE,), f32),
        pltpu.VMEM_SHARED((ACC_R, D), f32),
        pltpu.VMEM_SHARED((NS * HALF,), f32),
        pltpu.VMEM_SHARED((NS * HALF,), i32),
        pltpu.SemaphoreType.DMA,
        pltpu.SemaphoreType.DMA,
        pltpu.SemaphoreType.DMA,
        pltpu.SemaphoreType.DMA,
        pltpu.SemaphoreType.DMA,
        pltpu.SemaphoreType.DMA,
        pltpu.SemaphoreType.DMA,
        pltpu.SemaphoreType.DMA,
        pltpu.SemaphoreType.DMA,
    ]
    mesh = plsc.VectorSubcoreMesh(core_axis_name="c", subcore_axis_name="s")
    fn = pl.kernel(_sc_body, out_type=out_type, mesh=mesh,
                   scratch_types=scratch,
                   compiler_params=pltpu.CompilerParams(
                       needs_layout_passes=False))
    return fn(y_pg, y_ps, y_gp, y_sp,
              ei_pg[0], ei_pg[1], ei_ps[0], ei_ps[1],
              ei_gp[0], ei_gp[1], ei_sp[0], ei_sp[1],
              ea_pg, ea_gp)


def _fin_body(spg, cpg, etspg, xg, wrpg, wepg, bepg, blpg,
              sgp, cgp, etsgp, xp, wrgp, wegp, begp, blgp,
              ssp, csp, wrsp, blsp,
              sps, cps, xs, wrps, blps,
              wlin, blin, pw,
              hp, gw, hs):
    f32 = jnp.float32

    def mean(sref, cref):
        return sref[...] / jnp.maximum(cref[...], 1.0)

    def etterm(etsref, weref, beref, cref):
        has = (cref[...] > 0.0).astype(f32)
        return (jnp.dot(etsref[...], weref[...], preferred_element_type=f32)
                + beref[...]) * has

    hgw = (mean(spg, cpg) + blpg[...]
           + jnp.dot(xg[...], wrpg[...], preferred_element_type=f32)
           + etterm(etspg, wepg, bepg, cpg))
    hgw = jnp.maximum(hgw, 0.0)
    g = jnp.dot(hgw, wlin[...], preferred_element_type=f32) + blin[...]
    gw[...] = jnp.where(g >= 0.0, g, pw[...] * g)

    hpf = (mean(sgp, cgp) + blgp[...]
           + jnp.dot(xp[...], wrgp[...], preferred_element_type=f32)
           + etterm(etsgp, wegp, begp, cgp)
           + mean(ssp, csp) + blsp[...]
           + jnp.dot(xp[...], wrsp[...], preferred_element_type=f32))
    hp[...] = jnp.maximum(hpf, 0.0)

    hsw = (mean(sps, cps) + blps[...]
           + jnp.dot(xs[...], wrps[...], preferred_element_type=f32))
    hs[...] = jnp.maximum(hsw, 0.0)


def _fin(spg, cpg, etspg, xg, wrpg, wepg, bepg, blpg,
         sgp, cgp, etsgp, xp, wrgp, wegp, begp, blgp,
         ssp, csp, wrsp, blsp,
         sps, cps, xs, wrps, blps,
         wlin, blin, pw):
    row = pl.BlockSpec((BLK, D), lambda i: (i, 0))
    col1 = pl.BlockSpec((BLK, 1), lambda i: (i, 0))
    ets = pl.BlockSpec((BLK, DE), lambda i: (i, 0))
    wdd = pl.BlockSpec((D, D), lambda i: (0, 0))
    wed = pl.BlockSpec((DE, D), lambda i: (0, 0))
    b1d = pl.BlockSpec((1, D), lambda i: (0, 0))
    wl = pl.BlockSpec((D, 1), lambda i: (0, 0))
    b11 = pl.BlockSpec((1, 1), lambda i: (0, 0))
    in_specs = [row, col1, ets, row, wdd, wed, b1d, b1d,
                row, col1, ets, row, wdd, wed, b1d, b1d,
                row, col1, wdd, b1d,
                row, col1, row, wdd, b1d,
                wl, b11, b11]
    out_specs = [row, col1, row]
    out_shape = [jax.ShapeDtypeStruct((N, D), jnp.float32),
                 jax.ShapeDtypeStruct((N, 1), jnp.float32),
                 jax.ShapeDtypeStruct((N, D), jnp.float32)]
    return pl.pallas_call(
        _fin_body, grid=(GRID,), in_specs=in_specs, out_specs=out_specs,
        out_shape=out_shape,
    )(spg, cpg, etspg, xg, wrpg, wepg, bepg, blpg,
      sgp, cgp, etsgp, xp, wrgp, wegp, begp, blgp,
      ssp, csp, wrsp, blsp,
      sps, cps, xs, wrps, blps,
      wlin, blin, pw)


def kernel(x_pfas, x_gw, x_sw,
           edge_index_pg, edge_index_gp, edge_index_ps, edge_index_sp,
           edge_attr_pg, edge_attr_gp,
           Wl_pg, bl_pg, Wr_pg, We_pg, be_pg,
           Wl_gp, bl_gp, Wr_gp, We_gp, be_gp,
           Wl_ps, bl_ps, Wr_ps,
           Wl_sp, bl_sp, Wr_sp,
           W_lin, b_lin, prelu_w):
    y_pg, y_ps, y_gp, y_sp = _pre(x_pfas, x_gw, x_sw,
                                  Wl_pg, Wl_ps, Wl_gp, Wl_sp)

    ea2_pg = edge_attr_pg.reshape(E * DE // D, D)
    ea2_gp = edge_attr_gp.reshape(E * DE // D, D)

    (s_pg, s_ps, s_gp, s_sp, c_pg, c_ps, c_gp, c_sp, ets_pg, ets_gp) = _sc(
        y_pg, y_ps, y_gp, y_sp,
        edge_index_pg, edge_index_ps, edge_index_gp, edge_index_sp,
        ea2_pg, ea2_gp)

    ets_pg = ets_pg.reshape(PAD_N, DE)
    ets_gp = ets_gp.reshape(PAD_N, DE)
    c_pg2 = c_pg.reshape(PAD_N, 1)
    c_ps2 = c_ps.reshape(PAD_N, 1)
    c_gp2 = c_gp.reshape(PAD_N, 1)
    c_sp2 = c_sp.reshape(PAD_N, 1)

    hp, gw, hs = _fin(
        s_pg, c_pg2, ets_pg, x_gw, Wr_pg, We_pg, be_pg.reshape(1, D),
        bl_pg.reshape(1, D),
        s_gp, c_gp2, ets_gp, x_pfas, Wr_gp, We_gp, be_gp.reshape(1, D),
        bl_gp.reshape(1, D),
        s_sp, c_sp2, Wr_sp, bl_sp.reshape(1, D),
        s_ps, c_ps2, x_sw, Wr_ps, bl_ps.reshape(1, D),
        W_lin, b_lin.reshape(1, 1), prelu_w.reshape(1, 1))

    return (hp, gw, hs)


# speedup vs baseline: 1.3977x; 1.1999x over previous
import jax
import jax.numpy as jnp
from jax import lax
from jax.experimental import pallas as pl
from jax.experimental.pallas import tpu as pltpu
from jax.experimental.pallas import tpu_sc as plsc

N = 10000
E = 320000
D = 128
DE = 16
L = 16
NS = 16
PAD_N = 10240
EC = E // NS
B = 64
RING = 4
HALF = PAD_N // 2
ACC_R = HALF + 64
HSLICE = HALF // NS
CHK = 2048
NCHK = (EC + CHK - 1) // CHK
TAIL = EC - (NCHK - 1) * CHK
VPC = CHK // L
ET_B = 64
BLK = 1000
GRID = N // BLK


def _pre_body(xp, xg, xs, wpg, wps, wgp, wsp, ypg, yps, ygp, ysp):
    f32 = jnp.float32
    ypg[...] = jnp.dot(xp[...], wpg[...], preferred_element_type=f32)
    yps[...] = jnp.dot(xp[...], wps[...], preferred_element_type=f32)
    ygp[...] = jnp.dot(xg[...], wgp[...], preferred_element_type=f32)
    ysp[...] = jnp.dot(xs[...], wsp[...], preferred_element_type=f32)


def _pre(xp, xg, xs, wpg, wps, wgp, wsp):
    row_spec = pl.BlockSpec((BLK, D), lambda i: (i, 0))
    w_spec = pl.BlockSpec((D, D), lambda i: (0, 0))
    return pl.pallas_call(
        _pre_body,
        grid=(GRID,),
        in_specs=[row_spec, row_spec, row_spec, w_spec, w_spec, w_spec,
                  w_spec],
        out_specs=[row_spec] * 4,
        out_shape=[jax.ShapeDtypeStruct((N, D), jnp.float32)] * 4,
    )(xp, xg, xs, wpg, wps, wgp, wsp)


def _sc_body(y_pg, y_ps, y_gp, y_sp,
             er_pg, ec_pg, er_ps, ec_ps, er_gp, ec_gp, er_sp, ec_sp,
             ea_pg, ea_gp,
             sum_pg, sum_ps, sum_gp, sum_sp, cnt_pg, cnt_ps, cnt_gp, cnt_sp,
             ets_pg, ets_gp,
             rowchk, colchk, rowbuf, rowidx, colbuf, pendrow, pendcol,
             cntloc, winloc, redcnt, redwin, cntred, win8buf, wmodbuf,
             adjbuf, etraw, etflat,
             acc, cntsh, winsh,
             crsem0, crsem1, ccsem0, ccsem1,
             gsem0, gsem1, gsem2, gsem3, ssem0, ssem1, ssem2, ssem3,
             esem):
    c = lax.axis_index("c")
    s = lax.axis_index("s")
    i32 = jnp.int32
    zf16 = jnp.zeros((L,), jnp.float32)
    of16 = jnp.ones((L,), jnp.float32)
    iota16 = lax.iota(i32, L)
    lo = c * HALF
    crsems = (crsem0, crsem1)
    ccsems = (ccsem0, ccsem1)
    gsems = (gsem0, gsem1, gsem2, gsem3)
    ssems = (ssem0, ssem1, ssem2, ssem3)

    def chunk_descs(er_hbm, ec_hbm, ch, p, sz=CHK):
        base2 = s * EC
        dr = pltpu.make_async_copy(
            er_hbm.at[pl.ds(base2 + ch * CHK, sz)],
            rowchk.at[pl.ds(p * CHK, sz)], crsems[p])
        dc = pltpu.make_async_copy(
            ec_hbm.at[pl.ds(base2 + ch * CHK, sz)],
            colchk.at[pl.ds(p * CHK, sz)], ccsems[p])
        return dr, dc

    def process(y_hbm, er_hbm, ec_hbm, sum_hbm, cnt_hbm, ea_hbm, ets_hbm):
        def zrow(r, _):
            for k in range(D // L):
                rowbuf[0, r, pl.ds(k * L, L)] = zf16
            return 0
        lax.fori_loop(0, B, zrow, 0)
        arows = ACC_R // NS
        a0 = s * arows
        for m in range(arows // B):
            pltpu.sync_copy(rowbuf.at[0], acc.at[pl.ds(a0 + m * B, B)])
        if arows % B:
            pltpu.sync_copy(
                rowbuf.at[0, pl.ds(0, arows % B)],
                acc.at[pl.ds(a0 + (arows // B) * B, arows % B)])

        m1_16 = jnp.full((L,), -1, i32)

        def initloc(i, _):
            cntloc[pl.ds(i * L, L)] = zf16
            winloc[pl.ds(i * L, L)] = m1_16
            return 0
        lax.fori_loop(0, HALF // L, initloc, 0)
        adjbuf[pl.ds(L, L)] = jnp.full((L,), -16, i32)

        plsc.subcore_barrier()

        def wait_scatter(k):
            pltpu.make_async_copy(
                rowbuf.at[k], acc.at[colbuf.at[k]], ssems[k]).wait()

        def consume_gather(k):
            pltpu.make_async_copy(
                y_hbm.at[rowidx.at[k]], rowbuf.at[k], gsems[k]).wait()
            pltpu.async_copy(
                rowbuf.at[k], acc.at[colbuf.at[k]], ssems[k], add=True)

        def do_fire(slot, f):
            @pl.when(f >= RING)
            def _():
                wait_scatter(slot)

            for k in range(B // L):
                colbuf[slot, pl.ds(k * L, L)] = pendcol[pl.ds(k * L, L)]
                rowidx[slot, pl.ds(k * L, L)] = pendrow[pl.ds(k * L, L)]
            pendrow[pl.ds(0, L)] = pendrow[pl.ds(B, L)]
            pendcol[pl.ds(0, L)] = pendcol[pl.ds(B, L)]
            pltpu.async_copy(
                y_hbm.at[rowidx.at[slot]], rowbuf.at[slot], gsems[slot])

            @pl.when(f >= 2)
            def _():
                consume_gather((slot + 2) % RING)

        def scan_vreg(off, e0, cnt, f):
            cvec = colchk[pl.ds(off, L)]
            rvec = rowchk[pl.ds(off, L)]
            cl = cvec - lo
            m = jnp.logical_and(cl >= 0, cl < HALF)
            cls = jnp.where(m, cl, 0)
            plsc.addupdate_scatter(cntloc, [cls], of16, mask=m)
            if ea_hbm is not None:
                key = jnp.where(m, cls * L + iota16,
                                jnp.full((L,), -16, i32))
                skey, sval = plsc.sort_key_val(key, iota16)
                adjbuf[pl.ds(0, L)] = skey
                nxt = adjbuf[pl.ds(1, L)]
                scol = skey >> 4
                winmask = jnp.logical_and(scol != (nxt >> 4), scol >= 0)
                evec = e0 + sval
                plsc.store_scatter(winloc, [jnp.maximum(scol, 0)], evec,
                                   mask=winmask)
            pcv = plsc.all_reduce_population_count(m)
            pc = pcv[0]
            plsc.store_compressed(pendrow.at[pl.ds(cnt, L)], rvec, mask=m)
            plsc.store_compressed(pendcol.at[pl.ds(cnt, L)], cls, mask=m)
            cnt2 = cnt + pc
            fire = cnt2 >= B

            for k in range(RING):
                @pl.when(jnp.logical_and(fire, (f % RING) == k))
                def _(k=k):
                    do_fire(k, f)

            cnt3 = jnp.where(fire, cnt2 - B, cnt2)
            f2 = jnp.where(fire, f + 1, f)
            return cnt3, f2

        def scan_chunk(p, ch, cnt, f):
            def vloop(v, carry):
                cnt_, f_ = carry
                e0 = s * EC + ch * CHK + v * L
                return scan_vreg(p * CHK + v * L, e0, cnt_, f_)
            return lax.fori_loop(0, VPC, vloop, (cnt, f))

        dr, dc = chunk_descs(er_hbm, ec_hbm, 0, 0)
        dr.start()
        dc.start()

        def pairloop(q, carry):
            cnt, f = carry
            ch = 2 * q
            d0r, d0c = chunk_descs(er_hbm, ec_hbm, ch, 0)
            d0r.wait()
            d0c.wait()
            d1r, d1c = chunk_descs(er_hbm, ec_hbm, ch + 1, 1)
            d1r.start()
            d1c.start()
            cnt, f = scan_chunk(0, ch, cnt, f)
            d1r.wait()
            d1c.wait()
            d2r, d2c = chunk_descs(er_hbm, ec_hbm, ch + 2, 0)
            d2r.start()
            d2c.start()
            cnt, f = scan_chunk(1, ch + 1, cnt, f)
            return cnt, f

        cnt, f = lax.fori_loop(0, NCHK // 2 - 1, pairloop,
                               (jnp.int32(0), jnp.int32(0)))

        d0r, d0c = chunk_descs(er_hbm, ec_hbm, NCHK - 2, 0)
        d0r.wait()
        d0c.wait()
        d1r, d1c = chunk_descs(er_hbm, ec_hbm, NCHK - 1, 1, sz=TAIL)
        d1r.start()
        d1c.start()
        cnt, f = scan_chunk(0, NCHK - 2, cnt, f)
        d1r.wait()
        d1c.wait()
        m1pad = jnp.full((L,), -1, i32)
        for t in range((CHK - TAIL) // L):
            colchk[pl.ds(CHK + TAIL + t * L, L)] = m1pad
        cnt, f = scan_chunk(1, NCHK - 1, cnt, f)

        for k in range(B // L):
            pos = iota16 + k * L
            mm = pos < cnt
            pendcol[pl.ds(k * L, L)] = jnp.where(
                mm, pendcol[pl.ds(k * L, L)], jnp.full((L,), HALF, i32))
            pendrow[pl.ds(k * L, L)] = jnp.where(
                mm, pendrow[pl.ds(k * L, L)], jnp.zeros((L,), i32))
        for k in range(RING):
            @pl.when((f % RING) == k)
            def _(k=k):
                do_fire(k, f)

        for k in range(RING):
            @pl.when(jnp.logical_and(f >= 1, ((f - 1) % RING) == k))
            def _(k=k):
                consume_gather(k)

        for k in range(RING):
            @pl.when((f % RING) == k)
            def _(k=k):
                consume_gather(k)

        for d in (3, 2, 1, 0):
            for k in range(RING):
                @pl.when(jnp.logical_and(f >= d, ((f - d) % RING) == k))
                def _(k=k):
                    wait_scatter(k)

        plsc.subcore_barrier()
        off = s * HSLICE
        pltpu.sync_copy(acc.at[pl.ds(off, HSLICE)],
                        sum_hbm.at[pl.ds(lo + off, HSLICE)])

        pltpu.sync_copy(cntloc, cntsh.at[pl.ds(s * HALF, HALF)])
        if ea_hbm is not None:
            pltpu.sync_copy(winloc, winsh.at[pl.ds(s * HALF, HALF)])
        plsc.subcore_barrier()

        cdescs = []
        wdescs = []
        for t in range(NS):
            dce = pltpu.make_async_copy(
                cntsh.at[pl.ds(t * HALF + off, HSLICE)],
                redcnt.at[pl.ds(t * HSLICE, HSLICE)], crsem0)
            dce.start()
            cdescs.append(dce)
            if ea_hbm is not None:
                dwe = pltpu.make_async_copy(
                    winsh.at[pl.ds(t * HALF + off, HSLICE)],
                    redwin.at[pl.ds(t * HSLICE, HSLICE)], crsem1)
                dwe.start()
                wdescs.append(dwe)
        for dce in cdescs:
            dce.wait()
        for dwe in wdescs:
            dwe.wait()

        def redbody(m, _):
            cv = redcnt[pl.ds(m * L, L)]
            for t in range(1, NS):
                cv = cv + redcnt[pl.ds(t * HSLICE + m * L, L)]
            cntred[pl.ds(m * L, L)] = cv
            if ea_hbm is not None:
                wv = redwin[pl.ds(m * L, L)]
                for t in range(1, NS):
                    wv = jnp.maximum(wv, redwin[pl.ds(t * HSLICE + m * L, L)])
                wv = jnp.maximum(wv, 0)
                win8buf[pl.ds(m * L, L)] = wv >> 3
                wmodbuf[pl.ds(m * L, L)] = (wv & 7) * DE
            return 0

        lax.fori_loop(0, HSLICE // L, redbody, 0)
        pltpu.sync_copy(cntred, cnt_hbm.at[pl.ds(lo + off, HSLICE)])

        if ea_hbm is not None:
            for bb in range(HSLICE // ET_B):
                pltpu.async_copy(
                    ea_hbm.at[win8buf.at[pl.ds(bb * ET_B, ET_B)]], etraw,
                    esem).wait()

                def etloop(i, _):
                    offv = wmodbuf[pl.ds(bb * ET_B + i, L)]
                    etflat[pl.ds(i * DE, L)] = etraw[i, pl.ds(offv[0], L)]
                    return 0

                lax.fori_loop(0, ET_B, etloop, 0)
                pltpu.sync_copy(
                    etflat,
                    ets_hbm.at[pl.ds((lo + off + bb * ET_B) * DE,
                                     ET_B * DE)])

        plsc.subcore_barrier()

    process(y_pg, er_pg, ec_pg, sum_pg, cnt_pg, ea_pg, ets_pg)
    process(y_gp, er_gp, ec_gp, sum_gp, cnt_gp, ea_gp, ets_gp)
    process(y_ps, er_ps, ec_ps, sum_ps, cnt_ps, None, None)
    process(y_sp, er_sp, ec_sp, sum_sp, cnt_sp, None, None)


def _sc(y_pg, y_ps, y_gp, y_sp, ei_pg, ei_ps, ei_gp, ei_sp, ea_pg, ea_gp):
    f32 = jnp.float32
    i32 = jnp.int32
    out_type = (
        [jax.ShapeDtypeStruct((PAD_N, D), f32)] * 4
        + [jax.ShapeDtypeStruct((PAD_N,), f32)] * 4
        + [jax.ShapeDtypeStruct((PAD_N * DE,), f32)] * 2
    )
    scratch = [
        pltpu.VMEM((2 * CHK,), i32),
        pltpu.VMEM((2 * CHK,), i32),
        pltpu.VMEM((RING, B, D), f32),
        pltpu.VMEM((RING, B), i32),
        pltpu.VMEM((RING, B), i32),
        pltpu.VMEM((B + L,), i32),
        pltpu.VMEM((B + L,), i32),
        pltpu.VMEM((HALF,), f32),
        pltpu.VMEM((HALF,), i32),
        pltpu.VMEM((NS * HSLICE,), f32),
        pltpu.VMEM((NS * HSLICE,), i32),
        pltpu.VMEM((HSLICE,), f32),
        pltpu.VMEM((HSLICE,), i32),
        pltpu.VMEM((HSLICE + L,), i32),
        pltpu.VMEM((2 * L,), i32),
        pltpu.VMEM((ET_B, D), f32),
        pltpu.VMEM((ET_B * DE,), f32),
        pltpu.VMEM_SHARED((ACC_R, D), f32),
        pltpu.VMEM_SHARED((NS * HALF,), f32),
        pltpu.VMEM_SHARED((NS * HALF,), i32),
        pltpu.SemaphoreType.DMA,
        pltpu.SemaphoreType.DMA,
        pltpu.SemaphoreType.DMA,
        pltpu.SemaphoreType.DMA,
        pltpu.SemaphoreType.DMA,
        pltpu.SemaphoreType.DMA,
        pltpu.SemaphoreType.DMA,
        pltpu.SemaphoreType.DMA,
        pltpu.SemaphoreType.DMA,
        pltpu.SemaphoreType.DMA,
        pltpu.SemaphoreType.DMA,
        pltpu.SemaphoreType.DMA,
        pltpu.SemaphoreType.DMA,
    ]
    mesh = plsc.VectorSubcoreMesh(core_axis_name="c", subcore_axis_name="s")
    fn = pl.kernel(_sc_body, out_type=out_type, mesh=mesh,
                   scratch_types=scratch,
                   compiler_params=pltpu.CompilerParams(
                       needs_layout_passes=False))
    return fn(y_pg, y_ps, y_gp, y_sp,
              ei_pg[0], ei_pg[1], ei_ps[0], ei_ps[1],
              ei_gp[0], ei_gp[1], ei_sp[0], ei_sp[1],
              ea_pg, ea_gp)


def _fin_body(spg, cpg, etspg, xg, wrpg, wepg, bepg, blpg,
              sgp, cgp, etsgp, xp, wrgp, wegp, begp, blgp,
              ssp, csp, wrsp, blsp,
              sps, cps, xs, wrps, blps,
              wlin, blin, pw,
              hp, gw, hs):
    f32 = jnp.float32

    def mean(sref, cref):
        return sref[...] / jnp.maximum(cref[...], 1.0)

    def etterm(etsref, weref, beref, cref):
        has = (cref[...] > 0.0).astype(f32)
        return (jnp.dot(etsref[...], weref[...], preferred_element_type=f32)
                + beref[...]) * has

    hgw = (mean(spg, cpg) + blpg[...]
           + jnp.dot(xg[...], wrpg[...], preferred_element_type=f32)
           + etterm(etspg, wepg, bepg, cpg))
    hgw = jnp.maximum(hgw, 0.0)
    g = jnp.dot(hgw, wlin[...], preferred_element_type=f32) + blin[...]
    gw[...] = jnp.where(g >= 0.0, g, pw[...] * g)

    hpf = (mean(sgp, cgp) + blgp[...]
           + jnp.dot(xp[...], wrgp[...], preferred_element_type=f32)
           + etterm(etsgp, wegp, begp, cgp)
           + mean(ssp, csp) + blsp[...]
           + jnp.dot(xp[...], wrsp[...], preferred_element_type=f32))
    hp[...] = jnp.maximum(hpf, 0.0)

    hsw = (mean(sps, cps) + blps[...]
           + jnp.dot(xs[...], wrps[...], preferred_element_type=f32))
    hs[...] = jnp.maximum(hsw, 0.0)


def _fin(spg, cpg, etspg, xg, wrpg, wepg, bepg, blpg,
         sgp, cgp, etsgp, xp, wrgp, wegp, begp, blgp,
         ssp, csp, wrsp, blsp,
         sps, cps, xs, wrps, blps,
         wlin, blin, pw):
    row = pl.BlockSpec((BLK, D), lambda i: (i, 0))
    col1 = pl.BlockSpec((BLK, 1), lambda i: (i, 0))
    ets = pl.BlockSpec((BLK, DE), lambda i: (i, 0))
    wdd = pl.BlockSpec((D, D), lambda i: (0, 0))
    wed = pl.BlockSpec((DE, D), lambda i: (0, 0))
    b1d = pl.BlockSpec((1, D), lambda i: (0, 0))
    wl = pl.BlockSpec((D, 1), lambda i: (0, 0))
    b11 = pl.BlockSpec((1, 1), lambda i: (0, 0))
    in_specs = [row, col1, ets, row, wdd, wed, b1d, b1d,
                row, col1, ets, row, wdd, wed, b1d, b1d,
                row, col1, wdd, b1d,
                row, col1, row, wdd, b1d,
                wl, b11, b11]
    out_specs = [row, col1, row]
    out_shape = [jax.ShapeDtypeStruct((N, D), jnp.float32),
                 jax.ShapeDtypeStruct((N, 1), jnp.float32),
                 jax.ShapeDtypeStruct((N, D), jnp.float32)]
    return pl.pallas_call(
        _fin_body, grid=(GRID,), in_specs=in_specs, out_specs=out_specs,
        out_shape=out_shape,
    )(spg, cpg, etspg, xg, wrpg, wepg, bepg, blpg,
      sgp, cgp, etsgp, xp, wrgp, wegp, begp, blgp,
      ssp, csp, wrsp, blsp,
      sps, cps, xs, wrps, blps,
      wlin, blin, pw)


def kernel(x_pfas, x_gw, x_sw,
           edge_index_pg, edge_index_gp, edge_index_ps, edge_index_sp,
           edge_attr_pg, edge_attr_gp,
           Wl_pg, bl_pg, Wr_pg, We_pg, be_pg,
           Wl_gp, bl_gp, Wr_gp, We_gp, be_gp,
           Wl_ps, bl_ps, Wr_ps,
           Wl_sp, bl_sp, Wr_sp,
           W_lin, b_lin, prelu_w):
    y_pg, y_ps, y_gp, y_sp = _pre(x_pfas, x_gw, x_sw,
                                  Wl_pg, Wl_ps, Wl_gp, Wl_sp)

    ea2_pg = edge_attr_pg.reshape(E * DE // D, D)
    ea2_gp = edge_attr_gp.reshape(E * DE // D, D)

    (s_pg, s_ps, s_gp, s_sp, c_pg, c_ps, c_gp, c_sp, ets_pg, ets_gp) = _sc(
        y_pg, y_ps, y_gp, y_sp,
        edge_index_pg, edge_index_ps, edge_index_gp, edge_index_sp,
        ea2_pg, ea2_gp)

    ets_pg = ets_pg.reshape(PAD_N, DE)
    ets_gp = ets_gp.reshape(PAD_N, DE)
    c_pg2 = c_pg.reshape(PAD_N, 1)
    c_ps2 = c_ps.reshape(PAD_N, 1)
    c_gp2 = c_gp.reshape(PAD_N, 1)
    c_sp2 = c_sp.reshape(PAD_N, 1)

    hp, gw, hs = _fin(
        s_pg, c_pg2, ets_pg, x_gw, Wr_pg, We_pg, be_pg.reshape(1, D),
        bl_pg.reshape(1, D),
        s_gp, c_gp2, ets_gp, x_pfas, Wr_gp, We_gp, be_gp.reshape(1, D),
        bl_gp.reshape(1, D),
        s_sp, c_sp2, Wr_sp, bl_sp.reshape(1, D),
        s_ps, c_ps2, x_sw, Wr_ps, bl_ps.reshape(1, D),
        W_lin, b_lin.reshape(1, 1), prelu_w.reshape(1, 1))

    return (hp, gw, hs)


# speedup vs baseline: 1.4031x; 1.0039x over previous
import jax
import jax.numpy as jnp
from jax import lax
from jax.experimental import pallas as pl
from jax.experimental.pallas import tpu as pltpu
from jax.experimental.pallas import tpu_sc as plsc

N = 10000
E = 320000
D = 128
DE = 16
L = 16
NS = 16
PAD_N = 10240
EC = E // NS
B = 64
RING = 5
CL = 3
HALF = PAD_N // 2
ACC_R = HALF + 64
HSLICE = HALF // NS
CHK = 1024
NCHK = (EC + CHK - 1) // CHK
TAIL = EC - (NCHK - 1) * CHK
VPC = CHK // L
ET_B = 32
BLK = 1000
GRID = N // BLK


def _pre_body(xp, xg, xs, wpg, wps, wgp, wsp, ypg, yps, ygp, ysp):
    f32 = jnp.float32
    ypg[...] = jnp.dot(xp[...], wpg[...], preferred_element_type=f32)
    yps[...] = jnp.dot(xp[...], wps[...], preferred_element_type=f32)
    ygp[...] = jnp.dot(xg[...], wgp[...], preferred_element_type=f32)
    ysp[...] = jnp.dot(xs[...], wsp[...], preferred_element_type=f32)


def _pre(xp, xg, xs, wpg, wps, wgp, wsp):
    row_spec = pl.BlockSpec((BLK, D), lambda i: (i, 0))
    w_spec = pl.BlockSpec((D, D), lambda i: (0, 0))
    return pl.pallas_call(
        _pre_body,
        grid=(GRID,),
        in_specs=[row_spec, row_spec, row_spec, w_spec, w_spec, w_spec,
                  w_spec],
        out_specs=[row_spec] * 4,
        out_shape=[jax.ShapeDtypeStruct((N, D), jnp.float32)] * 4,
    )(xp, xg, xs, wpg, wps, wgp, wsp)


def _sc_body(y_pg, y_ps, y_gp, y_sp,
             er_pg, ec_pg, er_ps, ec_ps, er_gp, ec_gp, er_sp, ec_sp,
             ea_pg, ea_gp,
             sum_pg, sum_ps, sum_gp, sum_sp, cnt_pg, cnt_ps, cnt_gp, cnt_sp,
             ets_pg, ets_gp,
             rowchk, colchk, rowbuf, rowidx, colbuf, pendrow, pendcol,
             cntloc, winloc, redcnt, redwin, cntred, win8buf, wmodbuf,
             adjbuf, etraw, etflat,
             acc, cntsh, winsh,
             crsem0, crsem1, ccsem0, ccsem1,
             gsem0, gsem1, gsem2, gsem3, gsem4,
             ssem0, ssem1, ssem2, ssem3, ssem4,
             esem):
    c = lax.axis_index("c")
    s = lax.axis_index("s")
    i32 = jnp.int32
    zf16 = jnp.zeros((L,), jnp.float32)
    of16 = jnp.ones((L,), jnp.float32)
    iota16 = lax.iota(i32, L)
    lo = c * HALF
    crsems = (crsem0, crsem1)
    ccsems = (ccsem0, ccsem1)
    gsems = (gsem0, gsem1, gsem2, gsem3, gsem4)
    ssems = (ssem0, ssem1, ssem2, ssem3, ssem4)

    def chunk_descs(eiref, ch, p, sz=CHK):
        er_hbm, ec_hbm = eiref
        base2 = s * EC
        dr = pltpu.make_async_copy(
            er_hbm.at[pl.ds(base2 + ch * CHK, sz)],
            rowchk.at[pl.ds(p * CHK, sz)], crsems[p])
        dc = pltpu.make_async_copy(
            ec_hbm.at[pl.ds(base2 + ch * CHK, sz)],
            colchk.at[pl.ds(p * CHK, sz)], ccsems[p])
        return dr, dc

    def process(y_hbm, ei_hbm, sum_hbm, cnt_hbm, ea_hbm, ets_hbm):
        def zrow(r, _):
            for k in range(D // L):
                rowbuf[0, r, pl.ds(k * L, L)] = zf16
            return 0
        lax.fori_loop(0, B, zrow, 0)
        arows = ACC_R // NS
        a0 = s * arows
        for m in range(arows // B):
            pltpu.sync_copy(rowbuf.at[0], acc.at[pl.ds(a0 + m * B, B)])
        if arows % B:
            pltpu.sync_copy(
                rowbuf.at[0, pl.ds(0, arows % B)],
                acc.at[pl.ds(a0 + (arows // B) * B, arows % B)])

        m1_16 = jnp.full((L,), -1, i32)

        def initloc(i, _):
            cntloc[pl.ds(i * L, L)] = zf16
            winloc[pl.ds(i * L, L)] = m1_16
            return 0
        lax.fori_loop(0, HALF // L, initloc, 0)
        adjbuf[pl.ds(L, L)] = jnp.full((L,), -16, i32)

        plsc.subcore_barrier()

        def wait_scatter(k):
            pltpu.make_async_copy(
                rowbuf.at[k], acc.at[colbuf.at[k]], ssems[k]).wait()

        def consume_gather(k):
            pltpu.make_async_copy(
                y_hbm.at[rowidx.at[k]], rowbuf.at[k], gsems[k]).wait()
            pltpu.async_copy(
                rowbuf.at[k], acc.at[colbuf.at[k]], ssems[k], add=True)

        def do_fire(slot, f):
            @pl.when(f >= RING)
            def _():
                wait_scatter(slot)

            for k in range(B // L):
                colbuf[slot, pl.ds(k * L, L)] = pendcol[pl.ds(k * L, L)]
                rowidx[slot, pl.ds(k * L, L)] = pendrow[pl.ds(k * L, L)]
            pendrow[pl.ds(0, L)] = pendrow[pl.ds(B, L)]
            pendcol[pl.ds(0, L)] = pendcol[pl.ds(B, L)]
            pltpu.async_copy(
                y_hbm.at[rowidx.at[slot]], rowbuf.at[slot], gsems[slot])

            @pl.when(f >= CL)
            def _():
                consume_gather((slot + RING - CL) % RING)

        def scan_vreg(off, e0, cnt, f):
            cvec = colchk[pl.ds(off, L)]
            rvec = rowchk[pl.ds(off, L)]
            cl = cvec - lo
            m = jnp.logical_and(cl >= 0, cl < HALF)
            cls = jnp.where(m, cl, 0)
            plsc.addupdate_scatter(cntloc, [cls], of16, mask=m)
            if ea_hbm is not None:
                key = jnp.where(m, cls * L + iota16,
                                jnp.full((L,), -16, i32))
                skey, sval = plsc.sort_key_val(key, iota16)
                adjbuf[pl.ds(0, L)] = skey
                nxt = adjbuf[pl.ds(1, L)]
                scol = skey >> 4
                winmask = jnp.logical_and(scol != (nxt >> 4), scol >= 0)
                evec = e0 + sval
                plsc.store_scatter(winloc, [jnp.maximum(scol, 0)], evec,
                                   mask=winmask)
            pcv = plsc.all_reduce_population_count(m)
            pc = pcv[0]
            plsc.store_compressed(pendrow.at[pl.ds(cnt, L)], rvec, mask=m)
            plsc.store_compressed(pendcol.at[pl.ds(cnt, L)], cls, mask=m)
            cnt2 = cnt + pc
            fire = cnt2 >= B

            for k in range(RING):
                @pl.when(jnp.logical_and(fire, (f % RING) == k))
                def _(k=k):
                    do_fire(k, f)

            cnt3 = jnp.where(fire, cnt2 - B, cnt2)
            f2 = jnp.where(fire, f + 1, f)
            return cnt3, f2

        def scan_chunk(p, ch, cnt, f):
            def vloop(v, carry):
                cnt_, f_ = carry
                e0 = s * EC + ch * CHK + v * L
                return scan_vreg(p * CHK + v * L, e0, cnt_, f_)
            return lax.fori_loop(0, VPC, vloop, (cnt, f))

        dr, dc = chunk_descs(ei_hbm, 0, 0)
        dr.start()
        dc.start()

        def pairloop(q, carry):
            cnt, f = carry
            ch = 2 * q
            d0r, d0c = chunk_descs(ei_hbm, ch, 0)
            d0r.wait()
            d0c.wait()
            d1r, d1c = chunk_descs(ei_hbm, ch + 1, 1)
            d1r.start()
            d1c.start()
            cnt, f = scan_chunk(0, ch, cnt, f)
            d1r.wait()
            d1c.wait()
            d2r, d2c = chunk_descs(ei_hbm, ch + 2, 0)
            d2r.start()
            d2c.start()
            cnt, f = scan_chunk(1, ch + 1, cnt, f)
            return cnt, f

        cnt, f = lax.fori_loop(0, NCHK // 2 - 1, pairloop,
                               (jnp.int32(0), jnp.int32(0)))

        d0r, d0c = chunk_descs(ei_hbm, NCHK - 2, 0)
        d0r.wait()
        d0c.wait()
        d1r, d1c = chunk_descs(ei_hbm, NCHK - 1, 1, sz=TAIL)
        d1r.start()
        d1c.start()
        cnt, f = scan_chunk(0, NCHK - 2, cnt, f)
        d1r.wait()
        d1c.wait()
        m1pad = jnp.full((L,), -1, i32)
        for t in range((CHK - TAIL) // L):
            colchk[pl.ds(CHK + TAIL + t * L, L)] = m1pad
        cnt, f = scan_chunk(1, NCHK - 1, cnt, f)

        for k in range(B // L):
            pos = iota16 + k * L
            mm = pos < cnt
            pendcol[pl.ds(k * L, L)] = jnp.where(
                mm, pendcol[pl.ds(k * L, L)], jnp.full((L,), HALF, i32))
            pendrow[pl.ds(k * L, L)] = jnp.where(
                mm, pendrow[pl.ds(k * L, L)], jnp.zeros((L,), i32))
        for k in range(RING):
            @pl.when((f % RING) == k)
            def _(k=k):
                do_fire(k, f)

        for d in range(CL - 1, -1, -1):
            for k in range(RING):
                @pl.when(jnp.logical_and(f >= d, ((f - d) % RING) == k))
                def _(k=k):
                    consume_gather(k)

        for d in range(RING - 1, -1, -1):
            for k in range(RING):
                @pl.when(jnp.logical_and(f >= d, ((f - d) % RING) == k))
                def _(k=k):
                    wait_scatter(k)

        plsc.subcore_barrier()
        off = s * HSLICE
        pltpu.sync_copy(acc.at[pl.ds(off, HSLICE)],
                        sum_hbm.at[pl.ds(lo + off, HSLICE)])

        pltpu.sync_copy(cntloc, cntsh.at[pl.ds(s * HALF, HALF)])
        if ea_hbm is not None:
            pltpu.sync_copy(winloc, winsh.at[pl.ds(s * HALF, HALF)])
        plsc.subcore_barrier()

        cdescs = []
        wdescs = []
        for t in range(NS):
            dce = pltpu.make_async_copy(
                cntsh.at[pl.ds(t * HALF + off, HSLICE)],
                redcnt.at[pl.ds(t * HSLICE, HSLICE)], crsem0)
            dce.start()
            cdescs.append(dce)
            if ea_hbm is not None:
                dwe = pltpu.make_async_copy(
                    winsh.at[pl.ds(t * HALF + off, HSLICE)],
                    redwin.at[pl.ds(t * HSLICE, HSLICE)], crsem1)
                dwe.start()
                wdescs.append(dwe)
        for dce in cdescs:
            dce.wait()
        for dwe in wdescs:
            dwe.wait()

        def redbody(m, _):
            cv = redcnt[pl.ds(m * L, L)]
            for t in range(1, NS):
                cv = cv + redcnt[pl.ds(t * HSLICE + m * L, L)]
            cntred[pl.ds(m * L, L)] = cv
            if ea_hbm is not None:
                wv = redwin[pl.ds(m * L, L)]
                for t in range(1, NS):
                    wv = jnp.maximum(wv, redwin[pl.ds(t * HSLICE + m * L, L)])
                wv = jnp.maximum(wv, 0)
                win8buf[pl.ds(m * L, L)] = wv >> 3
                wmodbuf[pl.ds(m * L, L)] = (wv & 7) * DE
            return 0

        lax.fori_loop(0, HSLICE // L, redbody, 0)
        pltpu.sync_copy(cntred, cnt_hbm.at[pl.ds(lo + off, HSLICE)])

        if ea_hbm is not None:
            for bb in range(HSLICE // ET_B):
                pltpu.async_copy(
                    ea_hbm.at[win8buf.at[pl.ds(bb * ET_B, ET_B)]], etraw,
                    esem).wait()

                def etloop(i, _):
                    offv = wmodbuf[pl.ds(bb * ET_B + i, L)]
                    etflat[pl.ds(i * DE, L)] = etraw[i, pl.ds(offv[0], L)]
                    return 0

                lax.fori_loop(0, ET_B, etloop, 0)
                pltpu.sync_copy(
                    etflat,
                    ets_hbm.at[pl.ds((lo + off + bb * ET_B) * DE,
                                     ET_B * DE)])

        plsc.subcore_barrier()

    process(y_pg, (er_pg, ec_pg), sum_pg, cnt_pg, ea_pg, ets_pg)
    process(y_gp, (er_gp, ec_gp), sum_gp, cnt_gp, ea_gp, ets_gp)
    process(y_ps, (er_ps, ec_ps), sum_ps, cnt_ps, None, None)
    process(y_sp, (er_sp, ec_sp), sum_sp, cnt_sp, None, None)


def _sc(y_pg, y_ps, y_gp, y_sp, ei_pg, ei_ps, ei_gp, ei_sp, ea_pg, ea_gp):
    f32 = jnp.float32
    i32 = jnp.int32
    out_type = (
        [jax.ShapeDtypeStruct((PAD_N, D), f32)] * 4
        + [jax.ShapeDtypeStruct((PAD_N,), f32)] * 4
        + [jax.ShapeDtypeStruct((PAD_N * DE,), f32)] * 2
    )
    scratch = [
        pltpu.VMEM((2 * CHK,), i32),
        pltpu.VMEM((2 * CHK,), i32),
        pltpu.VMEM((RING, B, D), f32),
        pltpu.VMEM((RING, B), i32),
        pltpu.VMEM((RING, B), i32),
        pltpu.VMEM((B + L,), i32),
        pltpu.VMEM((B + L,), i32),
        pltpu.VMEM((HALF,), f32),
        pltpu.VMEM((HALF,), i32),
        pltpu.VMEM((NS * HSLICE,), f32),
        pltpu.VMEM((NS * HSLICE,), i32),
        pltpu.VMEM((HSLICE,), f32),
        pltpu.VMEM((HSLICE,), i32),
        pltpu.VMEM((HSLICE + L,), i32),
        pltpu.VMEM((2 * L,), i32),
        pltpu.VMEM((ET_B, D), f32),
        pltpu.VMEM((ET_B * DE,), f32),
        pltpu.VMEM_SHARED((ACC_R, D), f32),
        pltpu.VMEM_SHARED((NS * HALF,), f32),
        pltpu.VMEM_SHARED((NS * HALF,), i32),
        pltpu.SemaphoreType.DMA,
        pltpu.SemaphoreType.DMA,
        pltpu.SemaphoreType.DMA,
        pltpu.SemaphoreType.DMA,
        pltpu.SemaphoreType.DMA,
        pltpu.SemaphoreType.DMA,
        pltpu.SemaphoreType.DMA,
        pltpu.SemaphoreType.DMA,
        pltpu.SemaphoreType.DMA,
        pltpu.SemaphoreType.DMA,
        pltpu.SemaphoreType.DMA,
        pltpu.SemaphoreType.DMA,
        pltpu.SemaphoreType.DMA,
        pltpu.SemaphoreType.DMA,
        pltpu.SemaphoreType.DMA,
    ]
    mesh = plsc.VectorSubcoreMesh(core_axis_name="c", subcore_axis_name="s")
    fn = pl.kernel(_sc_body, out_type=out_type, mesh=mesh,
                   scratch_types=scratch,
                   compiler_params=pltpu.CompilerParams(
                       needs_layout_passes=False))
    return fn(y_pg, y_ps, y_gp, y_sp,
              ei_pg[0], ei_pg[1], ei_ps[0], ei_ps[1],
              ei_gp[0], ei_gp[1], ei_sp[0], ei_sp[1],
              ea_pg, ea_gp)


def _fin_body(spg, cpg, etspg, xg, wrpg, wepg, bepg, blpg,
              sgp, cgp, etsgp, xp, wrgp, wegp, begp, blgp,
              ssp, csp, wrsp, blsp,
              sps, cps, xs, wrps, blps,
              wlin, blin, pw,
              hp, gw, hs):
    f32 = jnp.float32

    def mean(sref, cref):
        return sref[...] / jnp.maximum(cref[...], 1.0)

    def etterm(etsref, weref, beref, cref):
        has = (cref[...] > 0.0).astype(f32)
        return (jnp.dot(etsref[...], weref[...], preferred_element_type=f32)
                + beref[...]) * has

    hgw = (mean(spg, cpg) + blpg[...]
           + jnp.dot(xg[...], wrpg[...], preferred_element_type=f32)
           + etterm(etspg, wepg, bepg, cpg))
    hgw = jnp.maximum(hgw, 0.0)
    g = jnp.dot(hgw, wlin[...], preferred_element_type=f32) + blin[...]
    gw[...] = jnp.where(g >= 0.0, g, pw[...] * g)

    hpf = (mean(sgp, cgp) + blgp[...]
           + jnp.dot(xp[...], wrgp[...], preferred_element_type=f32)
           + etterm(etsgp, wegp, begp, cgp)
           + mean(ssp, csp) + blsp[...]
           + jnp.dot(xp[...], wrsp[...], preferred_element_type=f32))
    hp[...] = jnp.maximum(hpf, 0.0)

    hsw = (mean(sps, cps) + blps[...]
           + jnp.dot(xs[...], wrps[...], preferred_element_type=f32))
    hs[...] = jnp.maximum(hsw, 0.0)


def _fin(spg, cpg, etspg, xg, wrpg, wepg, bepg, blpg,
         sgp, cgp, etsgp, xp, wrgp, wegp, begp, blgp,
         ssp, csp, wrsp, blsp,
         sps, cps, xs, wrps, blps,
         wlin, blin, pw):
    row = pl.BlockSpec((BLK, D), lambda i: (i, 0))
    col1 = pl.BlockSpec((BLK, 1), lambda i: (i, 0))
    ets = pl.BlockSpec((BLK, DE), lambda i: (i, 0))
    wdd = pl.BlockSpec((D, D), lambda i: (0, 0))
    wed = pl.BlockSpec((DE, D), lambda i: (0, 0))
    b1d = pl.BlockSpec((1, D), lambda i: (0, 0))
    wl = pl.BlockSpec((D, 1), lambda i: (0, 0))
    b11 = pl.BlockSpec((1, 1), lambda i: (0, 0))
    in_specs = [row, col1, ets, row, wdd, wed, b1d, b1d,
                row, col1, ets, row, wdd, wed, b1d, b1d,
                row, col1, wdd, b1d,
                row, col1, row, wdd, b1d,
                wl, b11, b11]
    out_specs = [row, col1, row]
    out_shape = [jax.ShapeDtypeStruct((N, D), jnp.float32),
                 jax.ShapeDtypeStruct((N, 1), jnp.float32),
                 jax.ShapeDtypeStruct((N, D), jnp.float32)]
    return pl.pallas_call(
        _fin_body, grid=(GRID,), in_specs=in_specs, out_specs=out_specs,
        out_shape=out_shape,
    )(spg, cpg, etspg, xg, wrpg, wepg, bepg, blpg,
      sgp, cgp, etsgp, xp, wrgp, wegp, begp, blgp,
      ssp, csp, wrsp, blsp,
      sps, cps, xs, wrps, blps,
      wlin, blin, pw)


def kernel(x_pfas, x_gw, x_sw,
           edge_index_pg, edge_index_gp, edge_index_ps, edge_index_sp,
           edge_attr_pg, edge_attr_gp,
           Wl_pg, bl_pg, Wr_pg, We_pg, be_pg,
           Wl_gp, bl_gp, Wr_gp, We_gp, be_gp,
           Wl_ps, bl_ps, Wr_ps,
           Wl_sp, bl_sp, Wr_sp,
           W_lin, b_lin, prelu_w):
    y_pg, y_ps, y_gp, y_sp = _pre(x_pfas, x_gw, x_sw,
                                  Wl_pg, Wl_ps, Wl_gp, Wl_sp)

    ea2_pg = edge_attr_pg.reshape(E * DE // D, D)
    ea2_gp = edge_attr_gp.reshape(E * DE // D, D)

    (s_pg, s_ps, s_gp, s_sp, c_pg, c_ps, c_gp, c_sp, ets_pg, ets_gp) = _sc(
        y_pg, y_ps, y_gp, y_sp,
        edge_index_pg, edge_index_ps, edge_index_gp, edge_index_sp,
        ea2_pg, ea2_gp)

    ets_pg = ets_pg.reshape(PAD_N, DE)
    ets_gp = ets_gp.reshape(PAD_N, DE)
    c_pg2 = c_pg.reshape(PAD_N, 1)
    c_ps2 = c_ps.reshape(PAD_N, 1)
    c_gp2 = c_gp.reshape(PAD_N, 1)
    c_sp2 = c_sp.reshape(PAD_N, 1)

    hp, gw, hs = _fin(
        s_pg, c_pg2, ets_pg, x_gw, Wr_pg, We_pg, be_pg.reshape(1, D),
        bl_pg.reshape(1, D),
        s_gp, c_gp2, ets_gp, x_pfas, Wr_gp, We_gp, be_gp.reshape(1, D),
        bl_gp.reshape(1, D),
        s_sp, c_sp2, Wr_sp, bl_sp.reshape(1, D),
        s_ps, c_ps2, x_sw, Wr_ps, bl_ps.reshape(1, D),
        W_lin, b_lin.reshape(1, 1), prelu_w.reshape(1, 1))

    return (hp, gw, hs)


# speedup vs baseline: 1.4196x; 1.0118x over previous
import jax
import jax.numpy as jnp
from jax import lax
from jax.experimental import pallas as pl
from jax.experimental.pallas import tpu as pltpu
from jax.experimental.pallas import tpu_sc as plsc

N = 10000
E = 320000
D = 128
DE = 16
L = 16
NS = 16
PAD_N = 10240
EC = E // NS
B = 64
RING = 5
CL = 3
HALF = PAD_N // 2
ACC_R = HALF + 64
HSLICE = HALF // NS
CHK = 1024
NCHK = (EC + CHK - 1) // CHK
TAIL = EC - (NCHK - 1) * CHK
VPC = CHK // L
ET_B = 32
BLK = 1000
GRID = N // BLK


def _pre_body(xp, xg, xs, wpg, wps, wgp, wsp, ypg, yps, ygp, ysp):
    f32 = jnp.float32
    ypg[...] = jnp.dot(xp[...], wpg[...], preferred_element_type=f32)
    yps[...] = jnp.dot(xp[...], wps[...], preferred_element_type=f32)
    ygp[...] = jnp.dot(xg[...], wgp[...], preferred_element_type=f32)
    ysp[...] = jnp.dot(xs[...], wsp[...], preferred_element_type=f32)


def _pre(xp, xg, xs, wpg, wps, wgp, wsp):
    row_spec = pl.BlockSpec((BLK, D), lambda i: (i, 0))
    w_spec = pl.BlockSpec((D, D), lambda i: (0, 0))
    return pl.pallas_call(
        _pre_body,
        grid=(GRID,),
        in_specs=[row_spec, row_spec, row_spec, w_spec, w_spec, w_spec,
                  w_spec],
        out_specs=[row_spec] * 4,
        out_shape=[jax.ShapeDtypeStruct((N, D), jnp.float32)] * 4,
    )(xp, xg, xs, wpg, wps, wgp, wsp)


def _sc_body(y_pg, y_ps, y_gp, y_sp,
             er_pg, ec_pg, er_ps, ec_ps, er_gp, ec_gp, er_sp, ec_sp,
             ea_pg, ea_gp,
             sum_pg, sum_ps, sum_gp, sum_sp, cnt_pg, cnt_ps, cnt_gp, cnt_sp,
             ets_pg, ets_gp,
             rowchk, colchk, rowbuf, rowidx, colbuf, pendrow, pendcol,
             cntloc, winloc, redcnt, redwin, cntred, win8buf, wmodbuf,
             adjbuf, etraw, etflat,
             acc, cntsh, winsh,
             crsem0, crsem1, ccsem0, ccsem1,
             gsem0, gsem1, gsem2, gsem3, gsem4,
             ssem0, ssem1, ssem2, ssem3, ssem4,
             esem):
    c = lax.axis_index("c")
    s = lax.axis_index("s")
    i32 = jnp.int32
    zf16 = jnp.zeros((L,), jnp.float32)
    of16 = jnp.ones((L,), jnp.float32)
    iota16 = lax.iota(i32, L)
    lo = c * HALF
    crsems = (crsem0, crsem1)
    ccsems = (ccsem0, ccsem1)
    gsems = (gsem0, gsem1, gsem2, gsem3, gsem4)
    ssems = (ssem0, ssem1, ssem2, ssem3, ssem4)

    def chunk_descs(eiref, ch, p, sz=CHK):
        er_hbm, ec_hbm = eiref
        base2 = s * EC
        dr = pltpu.make_async_copy(
            er_hbm.at[pl.ds(base2 + ch * CHK, sz)],
            rowchk.at[pl.ds(p * CHK, sz)], crsems[p])
        dc = pltpu.make_async_copy(
            ec_hbm.at[pl.ds(base2 + ch * CHK, sz)],
            colchk.at[pl.ds(p * CHK, sz)], ccsems[p])
        return dr, dc

    def process(y_hbm, ei_hbm, sum_hbm, cnt_hbm, ea_hbm, ets_hbm):
        def zrow(r, _):
            for k in range(D // L):
                rowbuf[0, r, pl.ds(k * L, L)] = zf16
            return 0
        lax.fori_loop(0, B, zrow, 0)
        arows = ACC_R // NS
        a0 = s * arows
        zdescs = []
        for m in range(arows // B):
            zd = pltpu.make_async_copy(
                rowbuf.at[0], acc.at[pl.ds(a0 + m * B, B)], crsem0)
            zd.start()
            zdescs.append(zd)
        if arows % B:
            zd = pltpu.make_async_copy(
                rowbuf.at[0, pl.ds(0, arows % B)],
                acc.at[pl.ds(a0 + (arows // B) * B, arows % B)], crsem0)
            zd.start()
            zdescs.append(zd)
        for zd in zdescs:
            zd.wait()

        m1_16 = jnp.full((L,), -1, i32)

        def initloc(i, _):
            cntloc[pl.ds(i * L, L)] = zf16
            winloc[pl.ds(i * L, L)] = m1_16
            return 0
        lax.fori_loop(0, HALF // L, initloc, 0)
        adjbuf[pl.ds(L, L)] = jnp.full((L,), -16, i32)

        plsc.subcore_barrier()

        def wait_scatter(k):
            pltpu.make_async_copy(
                rowbuf.at[k], acc.at[colbuf.at[k]], ssems[k]).wait()

        def consume_gather(k):
            pltpu.make_async_copy(
                y_hbm.at[rowidx.at[k]], rowbuf.at[k], gsems[k]).wait()
            pltpu.async_copy(
                rowbuf.at[k], acc.at[colbuf.at[k]], ssems[k], add=True)

        def do_fire(slot, f):
            @pl.when(f >= RING)
            def _():
                wait_scatter(slot)

            for k in range(B // L):
                colbuf[slot, pl.ds(k * L, L)] = pendcol[pl.ds(k * L, L)]
                rowidx[slot, pl.ds(k * L, L)] = pendrow[pl.ds(k * L, L)]
            pendrow[pl.ds(0, L)] = pendrow[pl.ds(B, L)]
            pendcol[pl.ds(0, L)] = pendcol[pl.ds(B, L)]
            pltpu.async_copy(
                y_hbm.at[rowidx.at[slot]], rowbuf.at[slot], gsems[slot])

            @pl.when(f >= CL)
            def _():
                consume_gather((slot + RING - CL) % RING)

        def scan_vreg(off, e0, cnt, f):
            cvec = colchk[pl.ds(off, L)]
            rvec = rowchk[pl.ds(off, L)]
            cl = cvec - lo
            m = jnp.logical_and(cl >= 0, cl < HALF)
            cls = jnp.where(m, cl, 0)
            plsc.addupdate_scatter(cntloc, [cls], of16, mask=m)
            if ea_hbm is not None:
                key = jnp.where(m, cls * L + iota16,
                                jnp.full((L,), -16, i32))
                skey, sval = plsc.sort_key_val(key, iota16)
                adjbuf[pl.ds(0, L)] = skey
                nxt = adjbuf[pl.ds(1, L)]
                scol = skey >> 4
                winmask = jnp.logical_and(scol != (nxt >> 4), scol >= 0)
                evec = e0 + sval
                plsc.store_scatter(winloc, [jnp.maximum(scol, 0)], evec,
                                   mask=winmask)
            pcv = plsc.all_reduce_population_count(m)
            pc = pcv[0]
            plsc.store_compressed(pendrow.at[pl.ds(cnt, L)], rvec, mask=m)
            plsc.store_compressed(pendcol.at[pl.ds(cnt, L)], cls, mask=m)
            cnt2 = cnt + pc
            fire = cnt2 >= B

            for k in range(RING):
                @pl.when(jnp.logical_and(fire, (f % RING) == k))
                def _(k=k):
                    do_fire(k, f)

            cnt3 = jnp.where(fire, cnt2 - B, cnt2)
            f2 = jnp.where(fire, f + 1, f)
            return cnt3, f2

        def scan_chunk(p, ch, cnt, f):
            def vloop(v, carry):
                cnt_, f_ = carry
                e0 = s * EC + ch * CHK + v * L
                return scan_vreg(p * CHK + v * L, e0, cnt_, f_)
            return lax.fori_loop(0, VPC, vloop, (cnt, f))

        dr, dc = chunk_descs(ei_hbm, 0, 0)
        dr.start()
        dc.start()

        def pairloop(q, carry):
            cnt, f = carry
            ch = 2 * q
            d0r, d0c = chunk_descs(ei_hbm, ch, 0)
            d0r.wait()
            d0c.wait()
            d1r, d1c = chunk_descs(ei_hbm, ch + 1, 1)
            d1r.start()
            d1c.start()
            cnt, f = scan_chunk(0, ch, cnt, f)
            d1r.wait()
            d1c.wait()
            d2r, d2c = chunk_descs(ei_hbm, ch + 2, 0)
            d2r.start()
            d2c.start()
            cnt, f = scan_chunk(1, ch + 1, cnt, f)
            return cnt, f

        cnt, f = lax.fori_loop(0, NCHK // 2 - 1, pairloop,
                               (jnp.int32(0), jnp.int32(0)))

        d0r, d0c = chunk_descs(ei_hbm, NCHK - 2, 0)
        d0r.wait()
        d0c.wait()
        d1r, d1c = chunk_descs(ei_hbm, NCHK - 1, 1, sz=TAIL)
        d1r.start()
        d1c.start()
        cnt, f = scan_chunk(0, NCHK - 2, cnt, f)
        d1r.wait()
        d1c.wait()
        m1pad = jnp.full((L,), -1, i32)
        for t in range((CHK - TAIL) // L):
            colchk[pl.ds(CHK + TAIL + t * L, L)] = m1pad
        cnt, f = scan_chunk(1, NCHK - 1, cnt, f)

        for k in range(B // L):
            pos = iota16 + k * L
            mm = pos < cnt
            pendcol[pl.ds(k * L, L)] = jnp.where(
                mm, pendcol[pl.ds(k * L, L)], jnp.full((L,), HALF, i32))
            pendrow[pl.ds(k * L, L)] = jnp.where(
                mm, pendrow[pl.ds(k * L, L)], jnp.zeros((L,), i32))
        for k in range(RING):
            @pl.when((f % RING) == k)
            def _(k=k):
                do_fire(k, f)

        for d in range(CL - 1, -1, -1):
            for k in range(RING):
                @pl.when(jnp.logical_and(f >= d, ((f - d) % RING) == k))
                def _(k=k):
                    consume_gather(k)

        for d in range(RING - 1, -1, -1):
            for k in range(RING):
                @pl.when(jnp.logical_and(f >= d, ((f - d) % RING) == k))
                def _(k=k):
                    wait_scatter(k)

        plsc.subcore_barrier()
        off = s * HSLICE
        pltpu.sync_copy(acc.at[pl.ds(off, HSLICE)],
                        sum_hbm.at[pl.ds(lo + off, HSLICE)])

        pltpu.sync_copy(cntloc, cntsh.at[pl.ds(s * HALF, HALF)])
        if ea_hbm is not None:
            pltpu.sync_copy(winloc, winsh.at[pl.ds(s * HALF, HALF)])
        plsc.subcore_barrier()

        cdescs = []
        wdescs = []
        for t in range(NS):
            dce = pltpu.make_async_copy(
                cntsh.at[pl.ds(t * HALF + off, HSLICE)],
                redcnt.at[pl.ds(t * HSLICE, HSLICE)], crsem0)
            dce.start()
            cdescs.append(dce)
            if ea_hbm is not None:
                dwe = pltpu.make_async_copy(
                    winsh.at[pl.ds(t * HALF + off, HSLICE)],
                    redwin.at[pl.ds(t * HSLICE, HSLICE)], crsem1)
                dwe.start()
                wdescs.append(dwe)
        for dce in cdescs:
            dce.wait()
        for dwe in wdescs:
            dwe.wait()

        def redbody(m, _):
            cv = redcnt[pl.ds(m * L, L)]
            for t in range(1, NS):
                cv = cv + redcnt[pl.ds(t * HSLICE + m * L, L)]
            cntred[pl.ds(m * L, L)] = cv
            if ea_hbm is not None:
                wv = redwin[pl.ds(m * L, L)]
                for t in range(1, NS):
                    wv = jnp.maximum(wv, redwin[pl.ds(t * HSLICE + m * L, L)])
                wv = jnp.maximum(wv, 0)
                win8buf[pl.ds(m * L, L)] = wv >> 3
                wmodbuf[pl.ds(m * L, L)] = (wv & 7) * DE
            return 0

        lax.fori_loop(0, HSLICE // L, redbody, 0)
        pltpu.sync_copy(cntred, cnt_hbm.at[pl.ds(lo + off, HSLICE)])

        if ea_hbm is not None:
            NBB = HSLICE // ET_B
            esems = (esem, crsem1)
            wsems = (ccsem0, ccsem1)

            def et_gather(bb, p):
                return pltpu.make_async_copy(
                    ea_hbm.at[win8buf.at[pl.ds(bb * ET_B, ET_B)]],
                    etraw.at[p], esems[p])

            def et_write(bb, p):
                return pltpu.make_async_copy(
                    etflat.at[pl.ds(p * ET_B * DE, ET_B * DE)],
                    ets_hbm.at[pl.ds((lo + off + bb * ET_B) * DE,
                                     ET_B * DE)], wsems[p])

            et_gather(0, 0).start()
            for bb in range(NBB):
                p = bb & 1
                et_gather(bb, p).wait()
                if bb + 1 < NBB:
                    et_gather(bb + 1, 1 - p).start()
                if bb >= 2:
                    et_write(bb - 2, p).wait()

                def etloop(i, _):
                    offv = wmodbuf[pl.ds(bb * ET_B + i, L)]
                    etflat[pl.ds((p * ET_B + i) * DE, L)] = (
                        etraw[p, i, pl.ds(offv[0], L)])
                    return 0

                lax.fori_loop(0, ET_B, etloop, 0)
                et_write(bb, p).start()
            et_write(NBB - 2, (NBB - 2) & 1).wait()
            et_write(NBB - 1, (NBB - 1) & 1).wait()

        plsc.subcore_barrier()

    process(y_pg, (er_pg, ec_pg), sum_pg, cnt_pg, ea_pg, ets_pg)
    process(y_gp, (er_gp, ec_gp), sum_gp, cnt_gp, ea_gp, ets_gp)
    process(y_ps, (er_ps, ec_ps), sum_ps, cnt_ps, None, None)
    process(y_sp, (er_sp, ec_sp), sum_sp, cnt_sp, None, None)


def _sc(y_pg, y_ps, y_gp, y_sp, ei_pg, ei_ps, ei_gp, ei_sp, ea_pg, ea_gp):
    f32 = jnp.float32
    i32 = jnp.int32
    out_type = (
        [jax.ShapeDtypeStruct((PAD_N, D), f32)] * 4
        + [jax.ShapeDtypeStruct((PAD_N,), f32)] * 4
        + [jax.ShapeDtypeStruct((PAD_N * DE,), f32)] * 2
    )
    scratch = [
        pltpu.VMEM((2 * CHK,), i32),
        pltpu.VMEM((2 * CHK,), i32),
        pltpu.VMEM((RING, B, D), f32),
        pltpu.VMEM((RING, B), i32),
        pltpu.VMEM((RING, B), i32),
        pltpu.VMEM((B + L,), i32),
        pltpu.VMEM((B + L,), i32),
        pltpu.VMEM((HALF,), f32),
        pltpu.VMEM((HALF,), i32),
        pltpu.VMEM((NS * HSLICE,), f32),
        pltpu.VMEM((NS * HSLICE,), i32),
        pltpu.VMEM((HSLICE,), f32),
        pltpu.VMEM((HSLICE,), i32),
        pltpu.VMEM((HSLICE + L,), i32),
        pltpu.VMEM((2 * L,), i32),
        pltpu.VMEM((2, ET_B, D), f32),
        pltpu.VMEM((2 * ET_B * DE,), f32),
        pltpu.VMEM_SHARED((ACC_R, D), f32),
        pltpu.VMEM_SHARED((NS * HALF,), f32),
        pltpu.VMEM_SHARED((NS * HALF,), i32),
        pltpu.SemaphoreType.DMA,
        pltpu.SemaphoreType.DMA,
        pltpu.SemaphoreType.DMA,
        pltpu.SemaphoreType.DMA,
        pltpu.SemaphoreType.DMA,
        pltpu.SemaphoreType.DMA,
        pltpu.SemaphoreType.DMA,
        pltpu.SemaphoreType.DMA,
        pltpu.SemaphoreType.DMA,
        pltpu.SemaphoreType.DMA,
        pltpu.SemaphoreType.DMA,
        pltpu.SemaphoreType.DMA,
        pltpu.SemaphoreType.DMA,
        pltpu.SemaphoreType.DMA,
        pltpu.SemaphoreType.DMA,
    ]
    mesh = plsc.VectorSubcoreMesh(core_axis_name="c", subcore_axis_name="s")
    fn = pl.kernel(_sc_body, out_type=out_type, mesh=mesh,
                   scratch_types=scratch,
                   compiler_params=pltpu.CompilerParams(
                       needs_layout_passes=False))
    return fn(y_pg, y_ps, y_gp, y_sp,
              ei_pg[0], ei_pg[1], ei_ps[0], ei_ps[1],
              ei_gp[0], ei_gp[1], ei_sp[0], ei_sp[1],
              ea_pg, ea_gp)


def _fin_body(spg, cpg, etspg, xg, wrpg, wepg, bepg, blpg,
              sgp, cgp, etsgp, xp, wrgp, wegp, begp, blgp,
              ssp, csp, wrsp, blsp,
              sps, cps, xs, wrps, blps,
              wlin, blin, pw,
              hp, gw, hs):
    f32 = jnp.float32

    def mean(sref, cref):
        return sref[...] / jnp.maximum(cref[...], 1.0)

    def etterm(etsref, weref, beref, cref):
        has = (cref[...] > 0.0).astype(f32)
        return (jnp.dot(etsref[...], weref[...], preferred_element_type=f32)
                + beref[...]) * has

    hgw = (mean(spg, cpg) + blpg[...]
           + jnp.dot(xg[...], wrpg[...], preferred_element_type=f32)
           + etterm(etspg, wepg, bepg, cpg))
    hgw = jnp.maximum(hgw, 0.0)
    g = jnp.dot(hgw, wlin[...], preferred_element_type=f32) + blin[...]
    gw[...] = jnp.where(g >= 0.0, g, pw[...] * g)

    hpf = (mean(sgp, cgp) + blgp[...]
           + jnp.dot(xp[...], wrgp[...], preferred_element_type=f32)
           + etterm(etsgp, wegp, begp, cgp)
           + mean(ssp, csp) + blsp[...]
           + jnp.dot(xp[...], wrsp[...], preferred_element_type=f32))
    hp[...] = jnp.maximum(hpf, 0.0)

    hsw = (mean(sps, cps) + blps[...]
           + jnp.dot(xs[...], wrps[...], preferred_element_type=f32))
    hs[...] = jnp.maximum(hsw, 0.0)


def _fin(spg, cpg, etspg, xg, wrpg, wepg, bepg, blpg,
         sgp, cgp, etsgp, xp, wrgp, wegp, begp, blgp,
         ssp, csp, wrsp, blsp,
         sps, cps, xs, wrps, blps,
         wlin, blin, pw):
    row = pl.BlockSpec((BLK, D), lambda i: (i, 0))
    col1 = pl.BlockSpec((BLK, 1), lambda i: (i, 0))
    ets = pl.BlockSpec((BLK, DE), lambda i: (i, 0))
    wdd = pl.BlockSpec((D, D), lambda i: (0, 0))
    wed = pl.BlockSpec((DE, D), lambda i: (0, 0))
    b1d = pl.BlockSpec((1, D), lambda i: (0, 0))
    wl = pl.BlockSpec((D, 1), lambda i: (0, 0))
    b11 = pl.BlockSpec((1, 1), lambda i: (0, 0))
    in_specs = [row, col1, ets, row, wdd, wed, b1d, b1d,
                row, col1, ets, row, wdd, wed, b1d, b1d,
                row, col1, wdd, b1d,
                row, col1, row, wdd, b1d,
                wl, b11, b11]
    out_specs = [row, col1, row]
    out_shape = [jax.ShapeDtypeStruct((N, D), jnp.float32),
                 jax.ShapeDtypeStruct((N, 1), jnp.float32),
                 jax.ShapeDtypeStruct((N, D), jnp.float32)]
    return pl.pallas_call(
        _fin_body, grid=(GRID,), in_specs=in_specs, out_specs=out_specs,
        out_shape=out_shape,
    )(spg, cpg, etspg, xg, wrpg, wepg, bepg, blpg,
      sgp, cgp, etsgp, xp, wrgp, wegp, begp, blgp,
      ssp, csp, wrsp, blsp,
      sps, cps, xs, wrps, blps,
      wlin, blin, pw)


def kernel(x_pfas, x_gw, x_sw,
           edge_index_pg, edge_index_gp, edge_index_ps, edge_index_sp,
           edge_attr_pg, edge_attr_gp,
           Wl_pg, bl_pg, Wr_pg, We_pg, be_pg,
           Wl_gp, bl_gp, Wr_gp, We_gp, be_gp,
           Wl_ps, bl_ps, Wr_ps,
           Wl_sp, bl_sp, Wr_sp,
           W_lin, b_lin, prelu_w):
    y_pg, y_ps, y_gp, y_sp = _pre(x_pfas, x_gw, x_sw,
                                  Wl_pg, Wl_ps, Wl_gp, Wl_sp)

    ea2_pg = edge_attr_pg.reshape(E * DE // D, D)
    ea2_gp = edge_attr_gp.reshape(E * DE // D, D)

    (s_pg, s_ps, s_gp, s_sp, c_pg, c_ps, c_gp, c_sp, ets_pg, ets_gp) = _sc(
        y_pg, y_ps, y_gp, y_sp,
        edge_index_pg, edge_index_ps, edge_index_gp, edge_index_sp,
        ea2_pg, ea2_gp)

    ets_pg = ets_pg.reshape(PAD_N, DE)
    ets_gp = ets_gp.reshape(PAD_N, DE)
    c_pg2 = c_pg.reshape(PAD_N, 1)
    c_ps2 = c_ps.reshape(PAD_N, 1)
    c_gp2 = c_gp.reshape(PAD_N, 1)
    c_sp2 = c_sp.reshape(PAD_N, 1)

    hp, gw, hs = _fin(
        s_pg, c_pg2, ets_pg, x_gw, Wr_pg, We_pg, be_pg.reshape(1, D),
        bl_pg.reshape(1, D),
        s_gp, c_gp2, ets_gp, x_pfas, Wr_gp, We_gp, be_gp.reshape(1, D),
        bl_gp.reshape(1, D),
        s_sp, c_sp2, Wr_sp, bl_sp.reshape(1, D),
        s_ps, c_ps2, x_sw, Wr_ps, bl_ps.reshape(1, D),
        W_lin, b_lin.reshape(1, 1), prelu_w.reshape(1, 1))

    return (hp, gw, hs)


# speedup vs baseline: 1.4382x; 1.0130x over previous
import jax
import jax.numpy as jnp
from jax import lax
from jax.experimental import pallas as pl
from jax.experimental.pallas import tpu as pltpu
from jax.experimental.pallas import tpu_sc as plsc

N = 10000
E = 320000
D = 128
DE = 16
L = 16
NS = 16
PAD_N = 10240
EC = E // NS
B = 64
RING = 5
CL = 3
HALF = PAD_N // 2
ACC_R = HALF + 64
HSLICE = HALF // NS
CHK = 1024
NCHK = (EC + CHK - 1) // CHK
TAIL = EC - (NCHK - 1) * CHK
VPC = CHK // L
ET_B = 32
BLK = 1000
GRID = N // BLK


def _pre_body(xp, xg, xs, wpg, wps, wgp, wsp, ypg, yps, ygp, ysp):
    f32 = jnp.float32
    ypg[...] = jnp.dot(xp[...], wpg[...], preferred_element_type=f32)
    yps[...] = jnp.dot(xp[...], wps[...], preferred_element_type=f32)
    ygp[...] = jnp.dot(xg[...], wgp[...], preferred_element_type=f32)
    ysp[...] = jnp.dot(xs[...], wsp[...], preferred_element_type=f32)


def _pre(xp, xg, xs, wpg, wps, wgp, wsp):
    row_spec = pl.BlockSpec((BLK, D), lambda i: (i, 0))
    w_spec = pl.BlockSpec((D, D), lambda i: (0, 0))
    return pl.pallas_call(
        _pre_body,
        grid=(GRID,),
        in_specs=[row_spec, row_spec, row_spec, w_spec, w_spec, w_spec,
                  w_spec],
        out_specs=[row_spec] * 4,
        out_shape=[jax.ShapeDtypeStruct((N, D), jnp.float32)] * 4,
    )(xp, xg, xs, wpg, wps, wgp, wsp)


def _sc_body(y_pg, y_ps, y_gp, y_sp,
             e_pg, e_ps, e_gp, e_sp,
             ea_pg, ea_gp,
             sum_pg, sum_ps, sum_gp, sum_sp, cnt_pg, cnt_ps, cnt_gp, cnt_sp,
             ets_pg, ets_gp,
             rowchk, colchk, rowbuf, rowidx, colbuf, pendrow, pendcol,
             cntloc, winloc, redcnt, redwin, cntred, win8buf, wmodbuf,
             adjbuf, etraw, etflat,
             acc, cntsh, winsh,
             crsem0, crsem1, ccsem0, ccsem1,
             gsem0, gsem1, gsem2, gsem3, gsem4,
             ssem0, ssem1, ssem2, ssem3, ssem4,
             esem):
    c = lax.axis_index("c")
    s = lax.axis_index("s")
    i32 = jnp.int32
    zf16 = jnp.zeros((L,), jnp.float32)
    of16 = jnp.ones((L,), jnp.float32)
    iota16 = lax.iota(i32, L)
    lo = c * HALF
    crsems = (crsem0, crsem1)
    ccsems = (ccsem0, ccsem1)
    gsems = (gsem0, gsem1, gsem2, gsem3, gsem4)
    ssems = (ssem0, ssem1, ssem2, ssem3, ssem4)

    def chunk_descs(e_hbm, ch, p, sz=CHK):
        base2 = s * EC
        dr = pltpu.make_async_copy(
            e_hbm.at[pl.ds(base2 + ch * CHK, sz)],
            rowchk.at[pl.ds(p * CHK, sz)], crsems[p])
        dc = pltpu.make_async_copy(
            e_hbm.at[pl.ds(E + base2 + ch * CHK, sz)],
            colchk.at[pl.ds(p * CHK, sz)], ccsems[p])
        return dr, dc

    def process(y_hbm, ei_hbm, sum_hbm, cnt_hbm, ea_hbm, ets_hbm):
        def zrow(r, _):
            for k in range(D // L):
                rowbuf[0, r, pl.ds(k * L, L)] = zf16
            return 0
        lax.fori_loop(0, B, zrow, 0)
        arows = ACC_R // NS
        a0 = s * arows
        zdescs = []
        for m in range(arows // B):
            zd = pltpu.make_async_copy(
                rowbuf.at[0], acc.at[pl.ds(a0 + m * B, B)], crsem0)
            zd.start()
            zdescs.append(zd)
        if arows % B:
            zd = pltpu.make_async_copy(
                rowbuf.at[0, pl.ds(0, arows % B)],
                acc.at[pl.ds(a0 + (arows // B) * B, arows % B)], crsem0)
            zd.start()
            zdescs.append(zd)
        for zd in zdescs:
            zd.wait()

        m1_16 = jnp.full((L,), -1, i32)

        def initloc(i, _):
            cntloc[pl.ds(i * L, L)] = zf16
            winloc[pl.ds(i * L, L)] = m1_16
            return 0
        lax.fori_loop(0, HALF // L, initloc, 0)
        adjbuf[pl.ds(L, L)] = jnp.full((L,), -16, i32)

        plsc.subcore_barrier()

        def wait_scatter(k):
            pltpu.make_async_copy(
                rowbuf.at[k], acc.at[colbuf.at[k]], ssems[k]).wait()

        def consume_gather(k):
            pltpu.make_async_copy(
                y_hbm.at[rowidx.at[k]], rowbuf.at[k], gsems[k]).wait()
            pltpu.async_copy(
                rowbuf.at[k], acc.at[colbuf.at[k]], ssems[k], add=True)

        def do_fire(slot, f):
            @pl.when(f >= RING)
            def _():
                wait_scatter(slot)

            for k in range(B // L):
                colbuf[slot, pl.ds(k * L, L)] = pendcol[pl.ds(k * L, L)]
                rowidx[slot, pl.ds(k * L, L)] = pendrow[pl.ds(k * L, L)]
            pendrow[pl.ds(0, L)] = pendrow[pl.ds(B, L)]
            pendcol[pl.ds(0, L)] = pendcol[pl.ds(B, L)]
            pltpu.async_copy(
                y_hbm.at[rowidx.at[slot]], rowbuf.at[slot], gsems[slot])

            @pl.when(f >= CL)
            def _():
                consume_gather((slot + RING - CL) % RING)

        def scan_vreg(off, e0, cnt, f):
            cvec = colchk[pl.ds(off, L)]
            rvec = rowchk[pl.ds(off, L)]
            cl = cvec - lo
            m = jnp.logical_and(cl >= 0, cl < HALF)
            cls = jnp.where(m, cl, 0)
            plsc.addupdate_scatter(cntloc, [cls], of16, mask=m)
            if ea_hbm is not None:
                key = jnp.where(m, cls * L + iota16,
                                jnp.full((L,), -16, i32))
                skey, sval = plsc.sort_key_val(key, iota16)
                adjbuf[pl.ds(0, L)] = skey
                nxt = adjbuf[pl.ds(1, L)]
                scol = skey >> 4
                winmask = jnp.logical_and(scol != (nxt >> 4), scol >= 0)
                evec = e0 + sval
                plsc.store_scatter(winloc, [jnp.maximum(scol, 0)], evec,
                                   mask=winmask)
            pcv = plsc.all_reduce_population_count(m)
            pc = pcv[0]
            plsc.store_compressed(pendrow.at[pl.ds(cnt, L)], rvec, mask=m)
            plsc.store_compressed(pendcol.at[pl.ds(cnt, L)], cls, mask=m)
            cnt2 = cnt + pc
            fire = cnt2 >= B

            for k in range(RING):
                @pl.when(jnp.logical_and(fire, (f % RING) == k))
                def _(k=k):
                    do_fire(k, f)

            cnt3 = jnp.where(fire, cnt2 - B, cnt2)
            f2 = jnp.where(fire, f + 1, f)
            return cnt3, f2

        def scan_chunk(p, ch, cnt, f):
            def vloop(v, carry):
                cnt_, f_ = carry
                e0 = s * EC + ch * CHK + v * L
                return scan_vreg(p * CHK + v * L, e0, cnt_, f_)
            return lax.fori_loop(0, VPC, vloop, (cnt, f))

        dr, dc = chunk_descs(ei_hbm, 0, 0)
        dr.start()
        dc.start()

        def pairloop(q, carry):
            cnt, f = carry
            ch = 2 * q
            d0r, d0c = chunk_descs(ei_hbm, ch, 0)
            d0r.wait()
            d0c.wait()
            d1r, d1c = chunk_descs(ei_hbm, ch + 1, 1)
            d1r.start()
            d1c.start()
            cnt, f = scan_chunk(0, ch, cnt, f)
            d1r.wait()
            d1c.wait()
            d2r, d2c = chunk_descs(ei_hbm, ch + 2, 0)
            d2r.start()
            d2c.start()
            cnt, f = scan_chunk(1, ch + 1, cnt, f)
            return cnt, f

        cnt, f = lax.fori_loop(0, NCHK // 2 - 1, pairloop,
                               (jnp.int32(0), jnp.int32(0)))

        d0r, d0c = chunk_descs(ei_hbm, NCHK - 2, 0)
        d0r.wait()
        d0c.wait()
        d1r, d1c = chunk_descs(ei_hbm, NCHK - 1, 1, sz=TAIL)
        d1r.start()
        d1c.start()
        cnt, f = scan_chunk(0, NCHK - 2, cnt, f)
        d1r.wait()
        d1c.wait()
        m1pad = jnp.full((L,), -1, i32)
        for t in range((CHK - TAIL) // L):
            colchk[pl.ds(CHK + TAIL + t * L, L)] = m1pad
        cnt, f = scan_chunk(1, NCHK - 1, cnt, f)

        for k in range(B // L):
            pos = iota16 + k * L
            mm = pos < cnt
            pendcol[pl.ds(k * L, L)] = jnp.where(
                mm, pendcol[pl.ds(k * L, L)], jnp.full((L,), HALF, i32))
            pendrow[pl.ds(k * L, L)] = jnp.where(
                mm, pendrow[pl.ds(k * L, L)], jnp.zeros((L,), i32))
        for k in range(RING):
            @pl.when((f % RING) == k)
            def _(k=k):
                do_fire(k, f)

        for d in range(CL - 1, -1, -1):
            for k in range(RING):
                @pl.when(jnp.logical_and(f >= d, ((f - d) % RING) == k))
                def _(k=k):
                    consume_gather(k)

        for d in range(RING - 1, -1, -1):
            for k in range(RING):
                @pl.when(jnp.logical_and(f >= d, ((f - d) % RING) == k))
                def _(k=k):
                    wait_scatter(k)

        plsc.subcore_barrier()
        off = s * HSLICE
        pltpu.sync_copy(acc.at[pl.ds(off, HSLICE)],
                        sum_hbm.at[pl.ds(lo + off, HSLICE)])

        pltpu.sync_copy(cntloc, cntsh.at[pl.ds(s * HALF, HALF)])
        if ea_hbm is not None:
            pltpu.sync_copy(winloc, winsh.at[pl.ds(s * HALF, HALF)])
        plsc.subcore_barrier()

        cdescs = []
        wdescs = []
        for t in range(NS):
            dce = pltpu.make_async_copy(
                cntsh.at[pl.ds(t * HALF + off, HSLICE)],
                redcnt.at[pl.ds(t * HSLICE, HSLICE)], crsem0)
            dce.start()
            cdescs.append(dce)
            if ea_hbm is not None:
                dwe = pltpu.make_async_copy(
                    winsh.at[pl.ds(t * HALF + off, HSLICE)],
                    redwin.at[pl.ds(t * HSLICE, HSLICE)], crsem1)
                dwe.start()
                wdescs.append(dwe)
        for dce in cdescs:
            dce.wait()
        for dwe in wdescs:
            dwe.wait()

        def redbody(m, _):
            cv = redcnt[pl.ds(m * L, L)]
            for t in range(1, NS):
                cv = cv + redcnt[pl.ds(t * HSLICE + m * L, L)]
            cntred[pl.ds(m * L, L)] = cv
            if ea_hbm is not None:
                wv = redwin[pl.ds(m * L, L)]
                for t in range(1, NS):
                    wv = jnp.maximum(wv, redwin[pl.ds(t * HSLICE + m * L, L)])
                wv = jnp.maximum(wv, 0)
                win8buf[pl.ds(m * L, L)] = wv >> 3
                wmodbuf[pl.ds(m * L, L)] = (wv & 7) * DE
            return 0

        lax.fori_loop(0, HSLICE // L, redbody, 0)
        pltpu.sync_copy(cntred, cnt_hbm.at[pl.ds(lo + off, HSLICE)])

        if ea_hbm is not None:
            NBB = HSLICE // ET_B
            esems = (esem, crsem1)
            wsems = (ccsem0, ccsem1)

            def et_gather(bb, p):
                return pltpu.make_async_copy(
                    ea_hbm.at[win8buf.at[pl.ds(bb * ET_B, ET_B)]],
                    etraw.at[p], esems[p])

            def et_write(bb, p):
                return pltpu.make_async_copy(
                    etflat.at[pl.ds(p * ET_B * DE, ET_B * DE)],
                    ets_hbm.at[pl.ds((lo + off + bb * ET_B) * DE,
                                     ET_B * DE)], wsems[p])

            et_gather(0, 0).start()
            for bb in range(NBB):
                p = bb & 1
                et_gather(bb, p).wait()
                if bb + 1 < NBB:
                    et_gather(bb + 1, 1 - p).start()
                if bb >= 2:
                    et_write(bb - 2, p).wait()

                def etloop(i, _):
                    offv = wmodbuf[pl.ds(bb * ET_B + i, L)]
                    etflat[pl.ds((p * ET_B + i) * DE, L)] = (
                        etraw[p, i, pl.ds(offv[0], L)])
                    return 0

                lax.fori_loop(0, ET_B, etloop, 0)
                et_write(bb, p).start()
            et_write(NBB - 2, (NBB - 2) & 1).wait()
            et_write(NBB - 1, (NBB - 1) & 1).wait()

        plsc.subcore_barrier()

    process(y_pg, e_pg, sum_pg, cnt_pg, ea_pg, ets_pg)
    process(y_gp, e_gp, sum_gp, cnt_gp, ea_gp, ets_gp)
    process(y_ps, e_ps, sum_ps, cnt_ps, None, None)
    process(y_sp, e_sp, sum_sp, cnt_sp, None, None)


def _sc(y_pg, y_ps, y_gp, y_sp, ei_pg, ei_ps, ei_gp, ei_sp, ea_pg, ea_gp):
    f32 = jnp.float32
    i32 = jnp.int32
    out_type = (
        [jax.ShapeDtypeStruct((PAD_N, D), f32)] * 4
        + [jax.ShapeDtypeStruct((PAD_N,), f32)] * 4
        + [jax.ShapeDtypeStruct((PAD_N * DE,), f32)] * 2
    )
    scratch = [
        pltpu.VMEM((2 * CHK,), i32),
        pltpu.VMEM((2 * CHK,), i32),
        pltpu.VMEM((RING, B, D), f32),
        pltpu.VMEM((RING, B), i32),
        pltpu.VMEM((RING, B), i32),
        pltpu.VMEM((B + L,), i32),
        pltpu.VMEM((B + L,), i32),
        pltpu.VMEM((HALF,), f32),
        pltpu.VMEM((HALF,), i32),
        pltpu.VMEM((NS * HSLICE,), f32),
        pltpu.VMEM((NS * HSLICE,), i32),
        pltpu.VMEM((HSLICE,), f32),
        pltpu.VMEM((HSLICE,), i32),
        pltpu.VMEM((HSLICE + L,), i32),
        pltpu.VMEM((2 * L,), i32),
        pltpu.VMEM((2, ET_B, D), f32),
        pltpu.VMEM((2 * ET_B * DE,), f32),
        pltpu.VMEM_SHARED((ACC_R, D), f32),
        pltpu.VMEM_SHARED((NS * HALF,), f32),
        pltpu.VMEM_SHARED((NS * HALF,), i32),
        pltpu.SemaphoreType.DMA,
        pltpu.SemaphoreType.DMA,
        pltpu.SemaphoreType.DMA,
        pltpu.SemaphoreType.DMA,
        pltpu.SemaphoreType.DMA,
        pltpu.SemaphoreType.DMA,
        pltpu.SemaphoreType.DMA,
        pltpu.SemaphoreType.DMA,
        pltpu.SemaphoreType.DMA,
        pltpu.SemaphoreType.DMA,
        pltpu.SemaphoreType.DMA,
        pltpu.SemaphoreType.DMA,
        pltpu.SemaphoreType.DMA,
        pltpu.SemaphoreType.DMA,
        pltpu.SemaphoreType.DMA,
    ]
    mesh = plsc.VectorSubcoreMesh(core_axis_name="c", subcore_axis_name="s")
    fn = pl.kernel(_sc_body, out_type=out_type, mesh=mesh,
                   scratch_types=scratch,
                   compiler_params=pltpu.CompilerParams(
                       needs_layout_passes=False))
    return fn(y_pg, y_ps, y_gp, y_sp,
              ei_pg.reshape(2 * E), ei_ps.reshape(2 * E),
              ei_gp.reshape(2 * E), ei_sp.reshape(2 * E),
              ea_pg, ea_gp)


def _fin_body(spg, cpg, etspg, xg, wrpg, wepg, bepg, blpg,
              sgp, cgp, etsgp, xp, wrgp, wegp, begp, blgp,
              ssp, csp, wrsp, blsp,
              sps, cps, xs, wrps, blps,
              wlin, blin, pw,
              hp, gw, hs):
    f32 = jnp.float32

    def mean(sref, cref):
        return sref[...] / jnp.maximum(cref[...], 1.0)

    def etterm(etsref, weref, beref, cref):
        has = (cref[...] > 0.0).astype(f32)
        return (jnp.dot(etsref[...], weref[...], preferred_element_type=f32)
                + beref[...]) * has

    hgw = (mean(spg, cpg) + blpg[...]
           + jnp.dot(xg[...], wrpg[...], preferred_element_type=f32)
           + etterm(etspg, wepg, bepg, cpg))
    hgw = jnp.maximum(hgw, 0.0)
    g = jnp.dot(hgw, wlin[...], preferred_element_type=f32) + blin[...]
    gw[...] = jnp.where(g >= 0.0, g, pw[...] * g)

    hpf = (mean(sgp, cgp) + blgp[...]
           + jnp.dot(xp[...], wrgp[...], preferred_element_type=f32)
           + etterm(etsgp, wegp, begp, cgp)
           + mean(ssp, csp) + blsp[...]
           + jnp.dot(xp[...], wrsp[...], preferred_element_type=f32))
    hp[...] = jnp.maximum(hpf, 0.0)

    hsw = (mean(sps, cps) + blps[...]
           + jnp.dot(xs[...], wrps[...], preferred_element_type=f32))
    hs[...] = jnp.maximum(hsw, 0.0)


def _fin(spg, cpg, etspg, xg, wrpg, wepg, bepg, blpg,
         sgp, cgp, etsgp, xp, wrgp, wegp, begp, blgp,
         ssp, csp, wrsp, blsp,
         sps, cps, xs, wrps, blps,
         wlin, blin, pw):
    row = pl.BlockSpec((BLK, D), lambda i: (i, 0))
    col1 = pl.BlockSpec((BLK, 1), lambda i: (i, 0))
    ets = pl.BlockSpec((BLK, DE), lambda i: (i, 0))
    wdd = pl.BlockSpec((D, D), lambda i: (0, 0))
    wed = pl.BlockSpec((DE, D), lambda i: (0, 0))
    b1d = pl.BlockSpec((1, D), lambda i: (0, 0))
    wl = pl.BlockSpec((D, 1), lambda i: (0, 0))
    b11 = pl.BlockSpec((1, 1), lambda i: (0, 0))
    in_specs = [row, col1, ets, row, wdd, wed, b1d, b1d,
                row, col1, ets, row, wdd, wed, b1d, b1d,
                row, col1, wdd, b1d,
                row, col1, row, wdd, b1d,
                wl, b11, b11]
    out_specs = [row, col1, row]
    out_shape = [jax.ShapeDtypeStruct((N, D), jnp.float32),
                 jax.ShapeDtypeStruct((N, 1), jnp.float32),
                 jax.ShapeDtypeStruct((N, D), jnp.float32)]
    return pl.pallas_call(
        _fin_body, grid=(GRID,), in_specs=in_specs, out_specs=out_specs,
        out_shape=out_shape,
    )(spg, cpg, etspg, xg, wrpg, wepg, bepg, blpg,
      sgp, cgp, etsgp, xp, wrgp, wegp, begp, blgp,
      ssp, csp, wrsp, blsp,
      sps, cps, xs, wrps, blps,
      wlin, blin, pw)


def kernel(x_pfas, x_gw, x_sw,
           edge_index_pg, edge_index_gp, edge_index_ps, edge_index_sp,
           edge_attr_pg, edge_attr_gp,
           Wl_pg, bl_pg, Wr_pg, We_pg, be_pg,
           Wl_gp, bl_gp, Wr_gp, We_gp, be_gp,
           Wl_ps, bl_ps, Wr_ps,
           Wl_sp, bl_sp, Wr_sp,
           W_lin, b_lin, prelu_w):
    y_pg, y_ps, y_gp, y_sp = _pre(x_pfas, x_gw, x_sw,
                                  Wl_pg, Wl_ps, Wl_gp, Wl_sp)

    ea2_pg = edge_attr_pg.reshape(E * DE // D, D)
    ea2_gp = edge_attr_gp.reshape(E * DE // D, D)

    (s_pg, s_ps, s_gp, s_sp, c_pg, c_ps, c_gp, c_sp, ets_pg, ets_gp) = _sc(
        y_pg, y_ps, y_gp, y_sp,
        edge_index_pg, edge_index_ps, edge_index_gp, edge_index_sp,
        ea2_pg, ea2_gp)

    ets_pg = ets_pg.reshape(PAD_N, DE)
    ets_gp = ets_gp.reshape(PAD_N, DE)
    c_pg2 = c_pg.reshape(PAD_N, 1)
    c_ps2 = c_ps.reshape(PAD_N, 1)
    c_gp2 = c_gp.reshape(PAD_N, 1)
    c_sp2 = c_sp.reshape(PAD_N, 1)

    hp, gw, hs = _fin(
        s_pg, c_pg2, ets_pg, x_gw, Wr_pg, We_pg, be_pg.reshape(1, D),
        bl_pg.reshape(1, D),
        s_gp, c_gp2, ets_gp, x_pfas, Wr_gp, We_gp, be_gp.reshape(1, D),
        bl_gp.reshape(1, D),
        s_sp, c_sp2, Wr_sp, bl_sp.reshape(1, D),
        s_ps, c_ps2, x_sw, Wr_ps, bl_ps.reshape(1, D),
        W_lin, b_lin.reshape(1, 1), prelu_w.reshape(1, 1))

    return (hp, gw, hs)


# speedup vs baseline: 1.4444x; 1.0044x over previous
import jax
import jax.numpy as jnp
from jax import lax
from jax.experimental import pallas as pl
from jax.experimental.pallas import tpu as pltpu
from jax.experimental.pallas import tpu_sc as plsc

N = 10000
E = 320000
D = 128
DE = 16
L = 16
NS = 16
PAD_N = 10240
EC = E // NS
B = 64
RING = 5
CL = 3
HALF = PAD_N // 2
ACC_R = HALF + 64
HSLICE = HALF // NS
CHK = 1024
NCHK = (EC + CHK - 1) // CHK
TAIL = EC - (NCHK - 1) * CHK
VPC = CHK // L
ET_B = 32
BLK = 1000
GRID = N // BLK


def _pre_body(xp, xg, xs, wpg, wps, wgp, wsp, ypg, yps, ygp, ysp):
    f32 = jnp.float32
    ypg[...] = jnp.dot(xp[...], wpg[...], preferred_element_type=f32)
    yps[...] = jnp.dot(xp[...], wps[...], preferred_element_type=f32)
    ygp[...] = jnp.dot(xg[...], wgp[...], preferred_element_type=f32)
    ysp[...] = jnp.dot(xs[...], wsp[...], preferred_element_type=f32)


def _pre(xp, xg, xs, wpg, wps, wgp, wsp):
    row_spec = pl.BlockSpec((BLK, D), lambda i: (i, 0))
    w_spec = pl.BlockSpec((D, D), lambda i: (0, 0))
    return pl.pallas_call(
        _pre_body,
        grid=(GRID,),
        in_specs=[row_spec, row_spec, row_spec, w_spec, w_spec, w_spec,
                  w_spec],
        out_specs=[row_spec] * 4,
        out_shape=[jax.ShapeDtypeStruct((N, D), jnp.float32)] * 4,
    )(xp, xg, xs, wpg, wps, wgp, wsp)


def _sc_body(y_pg, y_ps, y_gp, y_sp,
             e_pg, e_ps, e_gp, e_sp,
             ea_pg, ea_gp,
             sum_pg, sum_ps, sum_gp, sum_sp, cnt_pg, cnt_ps, cnt_gp, cnt_sp,
             ets_pg, ets_gp,
             rowchk, colchk, rowbuf, rowidx, colbuf, pendrow, pendcol,
             cntloc, winloc, redcnt, redwin, cntred, win8buf, wmodbuf,
             adjbuf, etraw, etflat,
             acc, cntsh, winsh,
             crsem0, crsem1, ccsem0, ccsem1,
             gsem0, gsem1, gsem2, gsem3, gsem4,
             ssem0, ssem1, ssem2, ssem3, ssem4,
             esem):
    c = lax.axis_index("c")
    s = lax.axis_index("s")
    i32 = jnp.int32
    zf16 = jnp.zeros((L,), jnp.float32)
    of16 = jnp.ones((L,), jnp.float32)
    iota16 = lax.iota(i32, L)
    idxp1 = jnp.minimum(iota16 + 1, L - 1)
    lane15 = iota16 == (L - 1)
    lo = c * HALF
    crsems = (crsem0, crsem1)
    ccsems = (ccsem0, ccsem1)
    gsems = (gsem0, gsem1, gsem2, gsem3, gsem4)
    ssems = (ssem0, ssem1, ssem2, ssem3, ssem4)

    def chunk_descs(e_hbm, ch, p, sz=CHK):
        base2 = s * EC
        dr = pltpu.make_async_copy(
            e_hbm.at[pl.ds(base2 + ch * CHK, sz)],
            rowchk.at[pl.ds(p * CHK, sz)], crsems[p])
        dc = pltpu.make_async_copy(
            e_hbm.at[pl.ds(E + base2 + ch * CHK, sz)],
            colchk.at[pl.ds(p * CHK, sz)], ccsems[p])
        return dr, dc

    def process(y_hbm, ei_hbm, sum_hbm, cnt_hbm, ea_hbm, ets_hbm):
        def zrow(r, _):
            for k in range(D // L):
                rowbuf[0, r, pl.ds(k * L, L)] = zf16
            return 0
        lax.fori_loop(0, B, zrow, 0)
        arows = ACC_R // NS
        a0 = s * arows
        zdescs = []
        for m in range(arows // B):
            zd = pltpu.make_async_copy(
                rowbuf.at[0], acc.at[pl.ds(a0 + m * B, B)], crsem0)
            zd.start()
            zdescs.append(zd)
        if arows % B:
            zd = pltpu.make_async_copy(
                rowbuf.at[0, pl.ds(0, arows % B)],
                acc.at[pl.ds(a0 + (arows // B) * B, arows % B)], crsem0)
            zd.start()
            zdescs.append(zd)
        for zd in zdescs:
            zd.wait()

        m1_16 = jnp.full((L,), -1, i32)

        def initloc(i, _):
            cntloc[pl.ds(i * L, L)] = zf16
            winloc[pl.ds(i * L, L)] = m1_16
            return 0
        lax.fori_loop(0, HALF // L, initloc, 0)
        adjbuf[pl.ds(L, L)] = jnp.full((L,), -16, i32)

        plsc.subcore_barrier()

        def wait_scatter(k):
            pltpu.make_async_copy(
                rowbuf.at[k], acc.at[colbuf.at[k]], ssems[k]).wait()

        def consume_gather(k):
            pltpu.make_async_copy(
                y_hbm.at[rowidx.at[k]], rowbuf.at[k], gsems[k]).wait()
            pltpu.async_copy(
                rowbuf.at[k], acc.at[colbuf.at[k]], ssems[k], add=True)

        def do_fire(slot, f):
            @pl.when(f >= RING)
            def _():
                wait_scatter(slot)

            for k in range(B // L):
                colbuf[slot, pl.ds(k * L, L)] = pendcol[pl.ds(k * L, L)]
                rowidx[slot, pl.ds(k * L, L)] = pendrow[pl.ds(k * L, L)]
            pendrow[pl.ds(0, L)] = pendrow[pl.ds(B, L)]
            pendcol[pl.ds(0, L)] = pendcol[pl.ds(B, L)]
            pltpu.async_copy(
                y_hbm.at[rowidx.at[slot]], rowbuf.at[slot], gsems[slot])

            @pl.when(f >= CL)
            def _():
                consume_gather((slot + RING - CL) % RING)

        def scan_vreg(off, e0, cnt, f):
            cvec = colchk[pl.ds(off, L)]
            rvec = rowchk[pl.ds(off, L)]
            cl = cvec - lo
            m = jnp.logical_and(cl >= 0, cl < HALF)
            cls = jnp.where(m, cl, 0)
            plsc.addupdate_scatter(cntloc, [cls], of16, mask=m)
            if ea_hbm is not None:
                key = jnp.where(m, cls * L + iota16,
                                jnp.full((L,), -16, i32))
                skey, sval = plsc.sort_key_val(key, iota16)
                nxt = skey.at[idxp1].get(mode="promise_in_bounds")
                scol = skey >> 4
                winmask = jnp.logical_and(
                    jnp.logical_or(scol != (nxt >> 4), lane15),
                    scol >= 0)
                evec = e0 + sval
                plsc.store_scatter(winloc, [jnp.maximum(scol, 0)], evec,
                                   mask=winmask)
            pcv = plsc.all_reduce_population_count(m)
            pc = pcv[0]
            plsc.store_compressed(pendrow.at[pl.ds(cnt, L)], rvec, mask=m)
            plsc.store_compressed(pendcol.at[pl.ds(cnt, L)], cls, mask=m)
            cnt2 = cnt + pc
            fire = cnt2 >= B

            for k in range(RING):
                @pl.when(jnp.logical_and(fire, (f % RING) == k))
                def _(k=k):
                    do_fire(k, f)

            cnt3 = jnp.where(fire, cnt2 - B, cnt2)
            f2 = jnp.where(fire, f + 1, f)
            return cnt3, f2

        def scan_chunk(p, ch, cnt, f):
            def vloop(v, carry):
                cnt_, f_ = carry
                e0 = s * EC + ch * CHK + v * L
                return scan_vreg(p * CHK + v * L, e0, cnt_, f_)
            return lax.fori_loop(0, VPC, vloop, (cnt, f))

        dr, dc = chunk_descs(ei_hbm, 0, 0)
        dr.start()
        dc.start()

        def pairloop(q, carry):
            cnt, f = carry
            ch = 2 * q
            d0r, d0c = chunk_descs(ei_hbm, ch, 0)
            d0r.wait()
            d0c.wait()
            d1r, d1c = chunk_descs(ei_hbm, ch + 1, 1)
            d1r.start()
            d1c.start()
            cnt, f = scan_chunk(0, ch, cnt, f)
            d1r.wait()
            d1c.wait()
            d2r, d2c = chunk_descs(ei_hbm, ch + 2, 0)
            d2r.start()
            d2c.start()
            cnt, f = scan_chunk(1, ch + 1, cnt, f)
            return cnt, f

        cnt, f = lax.fori_loop(0, NCHK // 2 - 1, pairloop,
                               (jnp.int32(0), jnp.int32(0)))

        d0r, d0c = chunk_descs(ei_hbm, NCHK - 2, 0)
        d0r.wait()
        d0c.wait()
        d1r, d1c = chunk_descs(ei_hbm, NCHK - 1, 1, sz=TAIL)
        d1r.start()
        d1c.start()
        cnt, f = scan_chunk(0, NCHK - 2, cnt, f)
        d1r.wait()
        d1c.wait()
        m1pad = jnp.full((L,), -1, i32)
        for t in range((CHK - TAIL) // L):
            colchk[pl.ds(CHK + TAIL + t * L, L)] = m1pad
        cnt, f = scan_chunk(1, NCHK - 1, cnt, f)

        for k in range(B // L):
            pos = iota16 + k * L
            mm = pos < cnt
            pendcol[pl.ds(k * L, L)] = jnp.where(
                mm, pendcol[pl.ds(k * L, L)], jnp.full((L,), HALF, i32))
            pendrow[pl.ds(k * L, L)] = jnp.where(
                mm, pendrow[pl.ds(k * L, L)], jnp.zeros((L,), i32))
        for k in range(RING):
            @pl.when((f % RING) == k)
            def _(k=k):
                do_fire(k, f)

        for d in range(CL - 1, -1, -1):
            for k in range(RING):
                @pl.when(jnp.logical_and(f >= d, ((f - d) % RING) == k))
                def _(k=k):
                    consume_gather(k)

        for d in range(RING - 1, -1, -1):
            for k in range(RING):
                @pl.when(jnp.logical_and(f >= d, ((f - d) % RING) == k))
                def _(k=k):
                    wait_scatter(k)

        plsc.subcore_barrier()
        off = s * HSLICE
        pltpu.sync_copy(acc.at[pl.ds(off, HSLICE)],
                        sum_hbm.at[pl.ds(lo + off, HSLICE)])

        pltpu.sync_copy(cntloc, cntsh.at[pl.ds(s * HALF, HALF)])
        if ea_hbm is not None:
            pltpu.sync_copy(winloc, winsh.at[pl.ds(s * HALF, HALF)])
        plsc.subcore_barrier()

        cdescs = []
        wdescs = []
        for t in range(NS):
            dce = pltpu.make_async_copy(
                cntsh.at[pl.ds(t * HALF + off, HSLICE)],
                redcnt.at[pl.ds(t * HSLICE, HSLICE)], crsem0)
            dce.start()
            cdescs.append(dce)
            if ea_hbm is not None:
                dwe = pltpu.make_async_copy(
                    winsh.at[pl.ds(t * HALF + off, HSLICE)],
                    redwin.at[pl.ds(t * HSLICE, HSLICE)], crsem1)
                dwe.start()
                wdescs.append(dwe)
        for dce in cdescs:
            dce.wait()
        for dwe in wdescs:
            dwe.wait()

        def redbody(m, _):
            cv = redcnt[pl.ds(m * L, L)]
            for t in range(1, NS):
                cv = cv + redcnt[pl.ds(t * HSLICE + m * L, L)]
            cntred[pl.ds(m * L, L)] = cv
            if ea_hbm is not None:
                wv = redwin[pl.ds(m * L, L)]
                for t in range(1, NS):
                    wv = jnp.maximum(wv, redwin[pl.ds(t * HSLICE + m * L, L)])
                wv = jnp.maximum(wv, 0)
                win8buf[pl.ds(m * L, L)] = wv >> 3
                wmodbuf[pl.ds(m * L, L)] = (wv & 7) * DE
            return 0

        lax.fori_loop(0, HSLICE // L, redbody, 0)
        pltpu.sync_copy(cntred, cnt_hbm.at[pl.ds(lo + off, HSLICE)])

        if ea_hbm is not None:
            NBB = HSLICE // ET_B
            esems = (esem, crsem1)
            wsems = (ccsem0, ccsem1)

            def et_gather(bb, p):
                return pltpu.make_async_copy(
                    ea_hbm.at[win8buf.at[pl.ds(bb * ET_B, ET_B)]],
                    etraw.at[p], esems[p])

            def et_write(bb, p):
                return pltpu.make_async_copy(
                    etflat.at[pl.ds(p * ET_B * DE, ET_B * DE)],
                    ets_hbm.at[pl.ds((lo + off + bb * ET_B) * DE,
                                     ET_B * DE)], wsems[p])

            et_gather(0, 0).start()
            for bb in range(NBB):
                p = bb & 1
                et_gather(bb, p).wait()
                if bb + 1 < NBB:
                    et_gather(bb + 1, 1 - p).start()
                if bb >= 2:
                    et_write(bb - 2, p).wait()

                def etloop(i, _):
                    offv = wmodbuf[pl.ds(bb * ET_B + i, L)]
                    etflat[pl.ds((p * ET_B + i) * DE, L)] = (
                        etraw[p, i, pl.ds(offv[0], L)])
                    return 0

                lax.fori_loop(0, ET_B, etloop, 0)
                et_write(bb, p).start()
            et_write(NBB - 2, (NBB - 2) & 1).wait()
            et_write(NBB - 1, (NBB - 1) & 1).wait()

        plsc.subcore_barrier()

    process(y_pg, e_pg, sum_pg, cnt_pg, ea_pg, ets_pg)
    process(y_gp, e_gp, sum_gp, cnt_gp, ea_gp, ets_gp)
    process(y_ps, e_ps, sum_ps, cnt_ps, None, None)
    process(y_sp, e_sp, sum_sp, cnt_sp, None, None)


def _sc(y_pg, y_ps, y_gp, y_sp, ei_pg, ei_ps, ei_gp, ei_sp, ea_pg, ea_gp):
    f32 = jnp.float32
    i32 = jnp.int32
    out_type = (
        [jax.ShapeDtypeStruct((PAD_N, D), f32)] * 4
        + [jax.ShapeDtypeStruct((PAD_N,), f32)] * 4
        + [jax.ShapeDtypeStruct((PAD_N * DE,), f32)] * 2
    )
    scratch = [
        pltpu.VMEM((2 * CHK,), i32),
        pltpu.VMEM((2 * CHK,), i32),
        pltpu.VMEM((RING, B, D), f32),
        pltpu.VMEM((RING, B), i32),
        pltpu.VMEM((RING, B), i32),
        pltpu.VMEM((B + L,), i32),
        pltpu.VMEM((B + L,), i32),
        pltpu.VMEM((HALF,), f32),
        pltpu.VMEM((HALF,), i32),
        pltpu.VMEM((NS * HSLICE,), f32),
        pltpu.VMEM((NS * HSLICE,), i32),
        pltpu.VMEM((HSLICE,), f32),
        pltpu.VMEM((HSLICE,), i32),
        pltpu.VMEM((HSLICE + L,), i32),
        pltpu.VMEM((2 * L,), i32),
        pltpu.VMEM((2, ET_B, D), f32),
        pltpu.VMEM((2 * ET_B * DE,), f32),
        pltpu.VMEM_SHARED((ACC_R, D), f32),
        pltpu.VMEM_SHARED((NS * HALF,), f32),
        pltpu.VMEM_SHARED((NS * HALF,), i32),
        pltpu.SemaphoreType.DMA,
        pltpu.SemaphoreType.DMA,
        pltpu.SemaphoreType.DMA,
        pltpu.SemaphoreType.DMA,
        pltpu.SemaphoreType.DMA,
        pltpu.SemaphoreType.DMA,
        pltpu.SemaphoreType.DMA,
        pltpu.SemaphoreType.DMA,
        pltpu.SemaphoreType.DMA,
        pltpu.SemaphoreType.DMA,
        pltpu.SemaphoreType.DMA,
        pltpu.SemaphoreType.DMA,
        pltpu.SemaphoreType.DMA,
        pltpu.SemaphoreType.DMA,
        pltpu.SemaphoreType.DMA,
    ]
    mesh = plsc.VectorSubcoreMesh(core_axis_name="c", subcore_axis_name="s")
    fn = pl.kernel(_sc_body, out_type=out_type, mesh=mesh,
                   scratch_types=scratch,
                   compiler_params=pltpu.CompilerParams(
                       needs_layout_passes=False))
    return fn(y_pg, y_ps, y_gp, y_sp,
              ei_pg.reshape(2 * E), ei_ps.reshape(2 * E),
              ei_gp.reshape(2 * E), ei_sp.reshape(2 * E),
              ea_pg, ea_gp)


def _fin_body(spg, cpg, etspg, xg, wrpg, wepg, bepg, blpg,
              sgp, cgp, etsgp, xp, wrgp, wegp, begp, blgp,
              ssp, csp, wrsp, blsp,
              sps, cps, xs, wrps, blps,
              wlin, blin, pw,
              hp, gw, hs):
    f32 = jnp.float32

    def mean(sref, cref):
        return sref[...] / jnp.maximum(cref[...], 1.0)

    def etterm(etsref, weref, beref, cref):
        has = (cref[...] > 0.0).astype(f32)
        return (jnp.dot(etsref[...], weref[...], preferred_element_type=f32)
                + beref[...]) * has

    hgw = (mean(spg, cpg) + blpg[...]
           + jnp.dot(xg[...], wrpg[...], preferred_element_type=f32)
           + etterm(etspg, wepg, bepg, cpg))
    hgw = jnp.maximum(hgw, 0.0)
    g = jnp.dot(hgw, wlin[...], preferred_element_type=f32) + blin[...]
    gw[...] = jnp.where(g >= 0.0, g, pw[...] * g)

    hpf = (mean(sgp, cgp) + blgp[...]
           + jnp.dot(xp[...], wrgp[...], preferred_element_type=f32)
           + etterm(etsgp, wegp, begp, cgp)
           + mean(ssp, csp) + blsp[...]
           + jnp.dot(xp[...], wrsp[...], preferred_element_type=f32))
    hp[...] = jnp.maximum(hpf, 0.0)

    hsw = (mean(sps, cps) + blps[...]
           + jnp.dot(xs[...], wrps[...], preferred_element_type=f32))
    hs[...] = jnp.maximum(hsw, 0.0)


def _fin(spg, cpg, etspg, xg, wrpg, wepg, bepg, blpg,
         sgp, cgp, etsgp, xp, wrgp, wegp, begp, blgp,
         ssp, csp, wrsp, blsp,
         sps, cps, xs, wrps, blps,
         wlin, blin, pw):
    row = pl.BlockSpec((BLK, D), lambda i: (i, 0))
    col1 = pl.BlockSpec((BLK, 1), lambda i: (i, 0))
    ets = pl.BlockSpec((BLK, DE), lambda i: (i, 0))
    wdd = pl.BlockSpec((D, D), lambda i: (0, 0))
    wed = pl.BlockSpec((DE, D), lambda i: (0, 0))
    b1d = pl.BlockSpec((1, D), lambda i: (0, 0))
    wl = pl.BlockSpec((D, 1), lambda i: (0, 0))
    b11 = pl.BlockSpec((1, 1), lambda i: (0, 0))
    in_specs = [row, col1, ets, row, wdd, wed, b1d, b1d,
                row, col1, ets, row, wdd, wed, b1d, b1d,
                row, col1, wdd, b1d,
                row, col1, row, wdd, b1d,
                wl, b11, b11]
    out_specs = [row, col1, row]
    out_shape = [jax.ShapeDtypeStruct((N, D), jnp.float32),
                 jax.ShapeDtypeStruct((N, 1), jnp.float32),
                 jax.ShapeDtypeStruct((N, D), jnp.float32)]
    return pl.pallas_call(
        _fin_body, grid=(GRID,), in_specs=in_specs, out_specs=out_specs,
        out_shape=out_shape,
    )(spg, cpg, etspg, xg, wrpg, wepg, bepg, blpg,
      sgp, cgp, etsgp, xp, wrgp, wegp, begp, blgp,
      ssp, csp, wrsp, blsp,
      sps, cps, xs, wrps, blps,
      wlin, blin, pw)


def kernel(x_pfas, x_gw, x_sw,
           edge_index_pg, edge_index_gp, edge_index_ps, edge_index_sp,
           edge_attr_pg, edge_attr_gp,
           Wl_pg, bl_pg, Wr_pg, We_pg, be_pg,
           Wl_gp, bl_gp, Wr_gp, We_gp, be_gp,
           Wl_ps, bl_ps, Wr_ps,
           Wl_sp, bl_sp, Wr_sp,
           W_lin, b_lin, prelu_w):
    y_pg, y_ps, y_gp, y_sp = _pre(x_pfas, x_gw, x_sw,
                                  Wl_pg, Wl_ps, Wl_gp, Wl_sp)

    ea2_pg = edge_attr_pg.reshape(E * DE // D, D)
    ea2_gp = edge_attr_gp.reshape(E * DE // D, D)

    (s_pg, s_ps, s_gp, s_sp, c_pg, c_ps, c_gp, c_sp, ets_pg, ets_gp) = _sc(
        y_pg, y_ps, y_gp, y_sp,
        edge_index_pg, edge_index_ps, edge_index_gp, edge_index_sp,
        ea2_pg, ea2_gp)

    ets_pg = ets_pg.reshape(PAD_N, DE)
    ets_gp = ets_gp.reshape(PAD_N, DE)
    c_pg2 = c_pg.reshape(PAD_N, 1)
    c_ps2 = c_ps.reshape(PAD_N, 1)
    c_gp2 = c_gp.reshape(PAD_N, 1)
    c_sp2 = c_sp.reshape(PAD_N, 1)

    hp, gw, hs = _fin(
        s_pg, c_pg2, ets_pg, x_gw, Wr_pg, We_pg, be_pg.reshape(1, D),
        bl_pg.reshape(1, D),
        s_gp, c_gp2, ets_gp, x_pfas, Wr_gp, We_gp, be_gp.reshape(1, D),
        bl_gp.reshape(1, D),
        s_sp, c_sp2, Wr_sp, bl_sp.reshape(1, D),
        s_ps, c_ps2, x_sw, Wr_ps, bl_ps.reshape(1, D),
        W_lin, b_lin.reshape(1, 1), prelu_w.reshape(1, 1))

    return (hp, gw, hs)
